# baseline (speedup 1.0000x reference)
import functools
import jax
import jax.numpy as jnp
from jax import lax
from jax.experimental import pallas as pl
from jax.experimental.pallas import tpu as pltpu
from jax.experimental.pallas import tpu_sc as plsc

N = 10000
E = 320000
H = 128
R = 1000
C = 5000

NB = 1000
N_PAD = 10240
C_PAD = 5120
E_PAD = 327680

NC = 2
NS = 16
L = 16
NW = NC * NS
EW = E_PAD // NW
ER = E_PAD // 128
WR = EW // 128
CHR = 16
NSEG = N_PAD // NS
CSEG = C_PAD // NS

_ZIDX = functools.partial(jnp.full, (L,), dtype=jnp.int32)


def _bcast0(ref):
    return plsc.load_gather(ref, [jnp.zeros((L,), jnp.int32)])


def _lk(x):
    return jnp.where(x >= 0, x, 0.01 * x)


def _nodes_body(x_ref, wh_ref, wt_ref, ah_ref, at_ref, xh_ref, xt_ref, nt_ref):
    x = x_ref[...]
    h = jax.nn.relu(jax.lax.dot(x, wh_ref[...], preferred_element_type=jnp.float32))
    t = jax.nn.relu(jax.lax.dot(x, wt_ref[...], preferred_element_type=jnp.float32))
    xh_ref[...] = h
    xt_ref[...] = t
    sh = jax.lax.dot_general(ah_ref[...], h, (((0,), (1,)), ((), ())),
                             preferred_element_type=jnp.float32)
    st = jax.lax.dot_general(at_ref[...], t, (((0,), (1,)), ((), ())),
                             preferred_element_type=jnp.float32)
    nrm = jnp.sqrt(jnp.sum(t * t, axis=1))[None, :]
    nt_ref[...] = jnp.concatenate([sh, st, nrm], axis=0).T


def _nodes_call(x_e, W_h, W_t, Ah, At):
    return pl.pallas_call(
        _nodes_body,
        grid=(N // NB,),
        in_specs=[
            pl.BlockSpec((NB, H), lambda i: (i, 0)),
            pl.BlockSpec((H, H), lambda i: (0, 0)),
            pl.BlockSpec((H, H), lambda i: (0, 0)),
            pl.BlockSpec((H, 4), lambda i: (0, 0)),
            pl.BlockSpec((H, 3), lambda i: (0, 0)),
        ],
        out_specs=[
            pl.BlockSpec((NB, H), lambda i: (i, 0)),
            pl.BlockSpec((NB, H), lambda i: (i, 0)),
            pl.BlockSpec((NB, 8), lambda i: (i, 0)),
        ],
        out_shape=[
            jax.ShapeDtypeStruct((N, H), jnp.float32),
            jax.ShapeDtypeStruct((N, H), jnp.float32),
            jax.ShapeDtypeStruct((N, 8), jnp.float32),
        ],
    )(x_e, W_h, W_t, Ah, At)


def _rel_body(nt_ref, re_ref, ar_ref, at2_ref, at3_ref, ac_ref, rt_ref, sv_ref,
              ntt_ref):
    rt = jax.lax.dot_general(ar_ref[...], re_ref[...], (((0,), (1,)), ((), ())),
                             preferred_element_type=jnp.float32)
    rt_ref[...] = rt
    nt = nt_ref[...].T
    ntt_ref[...] = nt
    mq1, mq2, mq3, mq4 = (jnp.max(nt[0]), jnp.max(nt[1]), jnp.max(nt[2]),
                          jnp.max(nt[3]))
    mt1 = jnp.max(nt[4])
    mr1 = jnp.max(rt[0])
    mr2 = jnp.max(rt[1])
    nt2 = jnp.sqrt(jnp.sum(at2_ref[...] ** 2))
    nt3 = jnp.sqrt(jnp.sum(at3_ref[...] ** 2))
    nac = jnp.sqrt(jnp.sum(ac_ref[...] ** 2))
    s1 = _lk((mq1 + mt1) / 2.0 + mr1)
    s2 = _lk(nt2 + mq2 + (mr2 + (mq3 + nt3) / 2.0) / 2.0)
    s3 = _lk(nac + mq4)
    sv = jnp.concatenate([jnp.stack([s1, s2, s3]), jnp.zeros((5,), jnp.float32)])
    sv_ref[...] = jnp.broadcast_to(sv[:, None], (8, 16))


def _rel_call(ntab, r_emb, Ar, a_t2, a_t3, a_c):
    return pl.pallas_call(
        _rel_body,
        in_specs=[
            pl.BlockSpec((N, 8), lambda: (0, 0)),
            pl.BlockSpec((R, H), lambda: (0, 0)),
            pl.BlockSpec((H, 2), lambda: (0, 0)),
            pl.BlockSpec((H,), lambda: (0,)),
            pl.BlockSpec((H,), lambda: (0,)),
            pl.BlockSpec((H,), lambda: (0,)),
        ],
        out_specs=[
            pl.BlockSpec((2, R), lambda: (0, 0)),
            pl.BlockSpec((8, 16), lambda: (0, 0)),
            pl.BlockSpec((8, N), lambda: (0, 0)),
        ],
        out_shape=[
            jax.ShapeDtypeStruct((2, R), jnp.float32),
            jax.ShapeDtypeStruct((8, 16), jnp.float32),
            jax.ShapeDtypeStruct((8, N), jnp.float32),
        ],
    )(ntab, r_emb, Ar, a_t2, a_t3, a_c)


def _edge1_body(eh_hbm, et_hbm, rel_hbm, tn_hbm, ntt_hbm, rtab_hbm, svec_hbm,
                u1_hbm, sumA_hbm,
                q1_v, t1_v, r1_v, sv_v, eh_v, et_v, rel_v, tn_v, u1_v, zz_v,
                sumA_sp):
    cid = lax.axis_index("c")
    sid = lax.axis_index("s")
    wid = sid * NC + cid
    pltpu.sync_copy(ntt_hbm.at[pl.ds(0, N)], q1_v)
    pltpu.sync_copy(ntt_hbm.at[pl.ds(4 * N, N)], t1_v)
    pltpu.sync_copy(rtab_hbm.at[pl.ds(0, R)], r1_v)
    pltpu.sync_copy(svec_hbm.at[pl.ds(0, L)], sv_v)

    def _zf(i, _):
        zz_v[pl.ds(i * L, L)] = jnp.zeros((L,), jnp.float32)
        return 0
    lax.fori_loop(0, NSEG // L, _zf, 0)
    pltpu.sync_copy(zz_v, sumA_sp.at[pl.ds(sid * NSEG, NSEG)])
    plsc.subcore_barrier()
    S1 = sv_v[...]
    base_row = wid * WR

    def _chunk(ch, _):
        r0 = base_row + ch * CHR
        pltpu.sync_copy(eh_hbm.at[pl.ds(r0, CHR)], eh_v)
        pltpu.sync_copy(et_hbm.at[pl.ds(r0, CHR)], et_v)
        pltpu.sync_copy(rel_hbm.at[pl.ds(r0, CHR)], rel_v)
        pltpu.sync_copy(tn_hbm.at[pl.ds(r0, CHR)], tn_v)

        def _row(r, _):
            for l in range(8):
                ehx = eh_v[r, pl.ds(l * L, L)]
                etx = et_v[r, pl.ds(l * L, L)]
                rlx = rel_v[r, pl.ds(l * L, L)]
                e1 = (plsc.load_gather(q1_v, [ehx])
                      + plsc.load_gather(t1_v, [etx])) * 0.5 \
                     + plsc.load_gather(r1_v, [rlx])
                z = jnp.where(e1 >= 0, e1, 0.01 * e1)
                u1_v[r, pl.ds(l * L, L)] = jnp.exp(z - S1)
            return 0
        lax.fori_loop(0, CHR, _row, 0)
        pltpu.sync_copy(u1_v, u1_hbm.at[pl.ds(r0, CHR)])

        def _srow(r, _):
            pltpu.sync_copy(u1_v.at[r], sumA_sp.at[tn_v.at[r]], add=True)
            return 0
        lax.fori_loop(0, CHR, _srow, 0)
        return 0
    lax.fori_loop(0, EW // (CHR * 128), _chunk, 0)
    plsc.subcore_barrier()
    pltpu.sync_copy(sumA_sp.at[pl.ds(sid * NSEG, NSEG)],
                    sumA_hbm.at[pl.ds(cid * N_PAD + sid * NSEG, NSEG)])


def _edge1_call(eh2, et2, rel2, tn2, ntt, rtab, svec):
    f = pl.kernel(
        _edge1_body,
        out_type=[
            jax.ShapeDtypeStruct((ER, 128), jnp.float32),
            jax.ShapeDtypeStruct((NC * N_PAD,), jnp.float32),
        ],
        mesh=plsc.VectorSubcoreMesh(core_axis_name="c", subcore_axis_name="s"),
        compiler_params=pltpu.CompilerParams(needs_layout_passes=False),
        scratch_types=[
            pltpu.VMEM((N,), jnp.float32),
            pltpu.VMEM((N,), jnp.float32),
            pltpu.VMEM((R,), jnp.float32),
            pltpu.VMEM((L,), jnp.float32),
            pltpu.VMEM((CHR, 128), jnp.int32),
            pltpu.VMEM((CHR, 128), jnp.int32),
            pltpu.VMEM((CHR, 128), jnp.int32),
            pltpu.VMEM((CHR, 128), jnp.int32),
            pltpu.VMEM((CHR, 128), jnp.float32),
            pltpu.VMEM((NSEG,), jnp.float32),
            pltpu.VMEM_SHARED((N_PAD,), jnp.float32),
        ],
    )
    return f(eh2, et2, rel2, tn2, ntt, rtab, svec)


def _edge2_body(eh_hbm, et_hbm, rel_hbm, tn_hbm, ci_hbm, u1_hbm,
                ntt_hbm, rtab_hbm, svec_hbm, sA_hbm,
                v_hbm, sumB_hbm,
                q2_v, q3_v, t2_v, t3_v, nrm_v, r2_v, sv_v, sa0_v, sa1_v,
                eh_v, et_v, rel_v, tn_v, ci_v, u1_v, u2_v, v_v, zz_v,
                sumB_sp):
    cid = lax.axis_index("c")
    sid = lax.axis_index("s")
    wid = sid * NC + cid
    pltpu.sync_copy(ntt_hbm.at[pl.ds(1 * N, N)], q2_v)
    pltpu.sync_copy(ntt_hbm.at[pl.ds(2 * N, N)], q3_v)
    pltpu.sync_copy(ntt_hbm.at[pl.ds(5 * N, N)], t2_v)
    pltpu.sync_copy(ntt_hbm.at[pl.ds(6 * N, N)], t3_v)
    pltpu.sync_copy(ntt_hbm.at[pl.ds(7 * N, N)], nrm_v)
    pltpu.sync_copy(rtab_hbm.at[pl.ds(R, R)], r2_v)
    pltpu.sync_copy(svec_hbm.at[pl.ds(L, L)], sv_v)
    pltpu.sync_copy(sA_hbm.at[pl.ds(0, N_PAD)], sa0_v)
    pltpu.sync_copy(sA_hbm.at[pl.ds(N_PAD, N_PAD)], sa1_v)

    def _zf(i, _):
        zz_v[pl.ds(i * L, L)] = jnp.zeros((L,), jnp.float32)
        return 0
    lax.fori_loop(0, NSEG // L, _zf, 0)

    @pl.when(sid < 8)
    def _zero_sumb():
        pltpu.sync_copy(zz_v, sumB_sp.at[pl.ds(sid * NSEG, NSEG)])
    plsc.subcore_barrier()

    S2 = sv_v[...]
    base_row = wid * WR

    def _chunk(ch, _):
        r0 = base_row + ch * CHR
        pltpu.sync_copy(eh_hbm.at[pl.ds(r0, CHR)], eh_v)
        pltpu.sync_copy(et_hbm.at[pl.ds(r0, CHR)], et_v)
        pltpu.sync_copy(rel_hbm.at[pl.ds(r0, CHR)], rel_v)
        pltpu.sync_copy(tn_hbm.at[pl.ds(r0, CHR)], tn_v)
        pltpu.sync_copy(ci_hbm.at[pl.ds(r0, CHR)], ci_v)
        pltpu.sync_copy(u1_hbm.at[pl.ds(r0, CHR)], u1_v)

        def _row(r, _):
            for l in range(8):
                ehx = eh_v[r, pl.ds(l * L, L)]
                etx = et_v[r, pl.ds(l * L, L)]
                rlx = rel_v[r, pl.ds(l * L, L)]
                tnx = tn_v[r, pl.ds(l * L, L)]
                u1x = u1_v[r, pl.ds(l * L, L)]
                sa = (plsc.load_gather(sa0_v, [tnx])
                      + plsc.load_gather(sa1_v, [tnx]))
                alpha = u1x / (sa + 1e-38)
                nr = plsc.load_gather(nrm_v, [etx])
                s = alpha / jnp.maximum(nr * alpha, 1e-12)
                e = (s * plsc.load_gather(t2_v, [etx])
                     + plsc.load_gather(q2_v, [ehx])
                     + (plsc.load_gather(r2_v, [rlx])
                        + (plsc.load_gather(q3_v, [ehx])
                           + s * plsc.load_gather(t3_v, [etx])) * 0.5) * 0.5)
                z = jnp.where(e >= 0, e, 0.01 * e)
                u2 = jnp.exp(z - S2)
                u2_v[r, pl.ds(l * L, L)] = u2
                v_v[r, pl.ds(l * L, L)] = s * u2
            return 0
        lax.fori_loop(0, CHR, _row, 0)
        pltpu.sync_copy(v_v, v_hbm.at[pl.ds(r0, CHR)])

        def _srow(r, _):
            pltpu.sync_copy(u2_v.at[r], sumB_sp.at[ci_v.at[r]], add=True)
            return 0
        lax.fori_loop(0, CHR, _srow, 0)
        return 0
    lax.fori_loop(0, EW // (CHR * 128), _chunk, 0)
    plsc.subcore_barrier()

    @pl.when(sid < 8)
    def _write_sumb():
        pltpu.sync_copy(sumB_sp.at[pl.ds(sid * NSEG, NSEG)],
                        sumB_hbm.at[pl.ds(cid * C_PAD + sid * NSEG, NSEG)])


def _edge2_call(eh2, et2, rel2, tn2, ci2, u1_2, ntt, rtab, svec, sumA_p):
    f = pl.kernel(
        _edge2_body,
        out_type=[
            jax.ShapeDtypeStruct((ER, 128), jnp.float32),
            jax.ShapeDtypeStruct((NC * C_PAD,), jnp.float32),
        ],
        mesh=plsc.VectorSubcoreMesh(core_axis_name="c", subcore_axis_name="s"),
        compiler_params=pltpu.CompilerParams(needs_layout_passes=False),
        scratch_types=[
            pltpu.VMEM((N,), jnp.float32),
            pltpu.VMEM((N,), jnp.float32),
            pltpu.VMEM((N,), jnp.float32),
            pltpu.VMEM((N,), jnp.float32),
            pltpu.VMEM((N,), jnp.float32),
            pltpu.VMEM((R,), jnp.float32),
            pltpu.VMEM((L,), jnp.float32),
            pltpu.VMEM((N_PAD,), jnp.float32),
            pltpu.VMEM((N_PAD,), jnp.float32),
            pltpu.VMEM((CHR, 128), jnp.int32),
            pltpu.VMEM((CHR, 128), jnp.int32),
            pltpu.VMEM((CHR, 128), jnp.int32),
            pltpu.VMEM((CHR, 128), jnp.int32),
            pltpu.VMEM((CHR, 128), jnp.int32),
            pltpu.VMEM((CHR, 128), jnp.float32),
            pltpu.VMEM((CHR, 128), jnp.float32),
            pltpu.VMEM((CHR, 128), jnp.float32),
            pltpu.VMEM((NSEG,), jnp.float32),
            pltpu.VMEM_SHARED((C_PAD,), jnp.float32),
        ],
    )
    return f(eh2, et2, rel2, tn2, ci2, u1_2, ntt, rtab, svec, sumA_p)


def _edge3_body(et_hbm, ci_hbm, v_hbm, xrt_hbm, sumB_hbm,
                acc_hbm,
                sb0_v, sb1_v, et_v, ci_v, v_v, w_v, rows_v, rows1_v, zz_v,
                acc_sp, sem):
    cid = lax.axis_index("c")
    sid = lax.axis_index("s")
    wid = sid * NC + cid
    pltpu.sync_copy(sumB_hbm.at[pl.ds(0, C_PAD)], sb0_v)
    pltpu.sync_copy(sumB_hbm.at[pl.ds(C_PAD, C_PAD)], sb1_v)

    def _zr(r, _):
        for l in range(8):
            zz_v[r, pl.ds(l * L, L)] = jnp.zeros((L,), jnp.float32)
        return 0
    lax.fori_loop(0, 16, _zr, 0)

    def _zc(i, _):
        pltpu.sync_copy(zz_v, acc_sp.at[pl.ds(sid * CSEG + i * 16, 16)])
        return 0
    lax.fori_loop(0, CSEG // 16, _zc, 0)
    plsc.subcore_barrier()

    base_row = wid * WR

    def _chunk(ch, _):
        r0 = base_row + ch * CHR
        pltpu.sync_copy(et_hbm.at[pl.ds(r0, CHR)], et_v)
        pltpu.sync_copy(ci_hbm.at[pl.ds(r0, CHR)], ci_v)
        pltpu.sync_copy(v_hbm.at[pl.ds(r0, CHR)], v_v)

        def _adj(r, _):
            for l in range(8):
                et_v[r, pl.ds(l * L, L)] = (et_v[r, pl.ds(l * L, L)]
                                            + cid * N)
            return 0
        lax.fori_loop(0, CHR, _adj, 0)

        def _wrow(r, _):
            for l in range(8):
                cix = ci_v[r, pl.ds(l * L, L)]
                sb = (plsc.load_gather(sb0_v, [cix])
                      + plsc.load_gather(sb1_v, [cix]))
                w_v[r, pl.ds(l * L, L)] = v_v[r, pl.ds(l * L, L)] / (sb + 1e-38)
            return 0
        lax.fori_loop(0, CHR, _wrow, 0)

        def _scale_row(rows_b, r):
            def _sg(g, _):
                for i in range(L):
                    e = g * L + i
                    wb = plsc.load_gather(
                        w_v, [jnp.full((L,), r, jnp.int32),
                              jnp.full((L,), e, jnp.int32)])
                    for l in range(8):
                        rows_b[e, pl.ds(l * L, L)] = (
                            rows_b[e, pl.ds(l * L, L)] * wb)
                return 0
            lax.fori_loop(0, 128 // L, _sg, 0)

        pltpu.async_copy(xrt_hbm.at[et_v.at[0]], rows_v, sem)

        def _pair(pr, _):
            r0 = 2 * pr
            r1 = 2 * pr + 1
            pltpu.async_copy(xrt_hbm.at[et_v.at[r1]], rows1_v, sem)
            pltpu.make_async_copy(xrt_hbm.at[et_v.at[r0]], rows_v, sem).wait()
            _scale_row(rows_v, r0)
            pltpu.sync_copy(rows_v, acc_sp.at[ci_v.at[r0]], add=True)

            @pl.when(pr < CHR // 2 - 1)
            def _fire_next():
                pltpu.async_copy(xrt_hbm.at[et_v.at[r0 + 2]], rows_v, sem)
            pltpu.make_async_copy(xrt_hbm.at[et_v.at[r1]], rows1_v, sem).wait()
            _scale_row(rows1_v, r1)
            pltpu.sync_copy(rows1_v, acc_sp.at[ci_v.at[r1]], add=True)
            return 0
        lax.fori_loop(0, CHR // 2, _pair, 0)
        return 0
    lax.fori_loop(0, EW // (CHR * 128), _chunk, 0)
    plsc.subcore_barrier()
    pltpu.sync_copy(acc_sp.at[pl.ds(sid * CSEG, CSEG)],
                    acc_hbm.at[pl.ds(cid * C_PAD + sid * CSEG, CSEG)])


def _edge3_call(et2, ci2, v2, x_r_t, sumB_p):
    f = pl.kernel(
        _edge3_body,
        out_type=[
            jax.ShapeDtypeStruct((NC * C_PAD, H), jnp.float32),
        ],
        mesh=plsc.VectorSubcoreMesh(core_axis_name="c", subcore_axis_name="s"),
        compiler_params=pltpu.CompilerParams(needs_layout_passes=False),
        scratch_types=[
            pltpu.VMEM((C_PAD,), jnp.float32),
            pltpu.VMEM((C_PAD,), jnp.float32),
            pltpu.VMEM((CHR, 128), jnp.int32),
            pltpu.VMEM((CHR, 128), jnp.int32),
            pltpu.VMEM((CHR, 128), jnp.float32),
            pltpu.VMEM((CHR, 128), jnp.float32),
            pltpu.VMEM((128, H), jnp.float32),
            pltpu.VMEM((128, H), jnp.float32),
            pltpu.VMEM((16, H), jnp.float32),
            pltpu.VMEM_SHARED((C_PAD, H), jnp.float32),
            pltpu.SemaphoreType.DMA,
        ],
    )
    return f(et2, ci2, v2, x_r_t, sumB_p)[0]


CW = C_PAD // NW
CWG = CW // L


def _cls1_body(acc_hbm, hc_hbm, ntt_hbm, ac_hbm, svec_hbm,
               xcls_hbm, u3_hbm, sumC_hbm,
               q4_v, ac_v, sv_v, hc_v, hc1_v, u3_v, u3f_v, ec_v, xc_v, xc1_v,
               zz_v, sumC_sp):
    cid = lax.axis_index("c")
    sid = lax.axis_index("s")
    wid = sid * NC + cid
    pltpu.sync_copy(ntt_hbm.at[pl.ds(3 * N, N)], q4_v)
    pltpu.sync_copy(ac_hbm, ac_v)
    pltpu.sync_copy(svec_hbm.at[pl.ds(2 * L, L)], sv_v)
    pltpu.sync_copy(hc_hbm.at[pl.ds(wid * 256, 256)], hc1_v)

    def _hcb(g, _):
        hc_v[g] = hc1_v[pl.ds(g * L, L)]
        return 0
    lax.fori_loop(0, CWG, _hcb, 0)

    def _zf(i, _):
        zz_v[pl.ds(i * L, L)] = jnp.zeros((L,), jnp.float32)
        return 0
    lax.fori_loop(0, NSEG // L, _zf, 0)
    pltpu.sync_copy(zz_v, sumC_sp.at[pl.ds(sid * NSEG, NSEG)])
    plsc.subcore_barrier()

    pltpu.sync_copy(acc_hbm.at[pl.ds(wid * CW, CW)], xc_v)
    pltpu.sync_copy(acc_hbm.at[pl.ds(C_PAD + wid * CW, CW)], xc1_v)

    def _addr(r, _):
        for l in range(8):
            xc_v[r, pl.ds(l * L, L)] = (xc_v[r, pl.ds(l * L, L)]
                                        + xc1_v[r, pl.ds(l * L, L)])
        return 0
    lax.fori_loop(0, CW, _addr, 0)
    pltpu.sync_copy(xc_v, xcls_hbm.at[pl.ds(wid * CW, CW)])

    def _dotg(g, _):
        rows16 = g * L + lax.broadcasted_iota(jnp.int32, (L,), 0)

        def _j(j, acc):
            colv = plsc.load_gather(xc_v, [rows16, jnp.full((L,), j, jnp.int32)])
            acb = plsc.load_gather(ac_v, [jnp.full((L,), j, jnp.int32)])
            return acc + colv * acb
        ec_v[g] = lax.fori_loop(0, H, _j, jnp.zeros((L,), jnp.float32))
        return 0
    lax.fori_loop(0, CWG, _dotg, 0)

    S3 = sv_v[...]

    def _grp(g, _):
        hcx = hc_v[g]
        q4g = plsc.load_gather(q4_v, [jnp.minimum(hcx, N - 1)])
        e_c = ec_v[g] + q4g
        z = jnp.where(e_c >= 0, e_c, 0.01 * e_c)
        u3g = jnp.where(hcx < N, jnp.exp(z - S3), 0.0)
        u3_v[g] = u3g
        u3f_v[pl.ds(g * L, L)] = u3g
        pltpu.sync_copy(u3_v.at[g], sumC_sp.at[hc_v.at[g]], add=True)
        return 0
    lax.fori_loop(0, CWG, _grp, 0)
    pltpu.sync_copy(u3f_v, u3_hbm.at[pl.ds(wid * 256, 256)])
    plsc.subcore_barrier()
    pltpu.sync_copy(sumC_sp.at[pl.ds(sid * NSEG, NSEG)],
                    sumC_hbm.at[pl.ds(cid * N_PAD + sid * NSEG, NSEG)])


def _cls1_call(acc_p, hc2, ntt, a_c, svec):
    f = pl.kernel(
        _cls1_body,
        out_type=[
            jax.ShapeDtypeStruct((C_PAD, H), jnp.float32),
            jax.ShapeDtypeStruct((NW * 256,), jnp.float32),
            jax.ShapeDtypeStruct((NC * N_PAD,), jnp.float32),
        ],
        mesh=plsc.VectorSubcoreMesh(core_axis_name="c", subcore_axis_name="s"),
        compiler_params=pltpu.CompilerParams(needs_layout_passes=False),
        scratch_types=[
            pltpu.VMEM((N,), jnp.float32),
            pltpu.VMEM((H,), jnp.float32),
            pltpu.VMEM((L,), jnp.float32),
            pltpu.VMEM((L, L), jnp.int32),
            pltpu.VMEM((256,), jnp.int32),
            pltpu.VMEM((L, L), jnp.float32),
            pltpu.VMEM((256,), jnp.float32),
            pltpu.VMEM((L, L), jnp.float32),
            pltpu.VMEM((CW, H), jnp.float32),
            pltpu.VMEM((CW, H), jnp.float32),
            pltpu.VMEM((NSEG,), jnp.float32),
            pltpu.VMEM_SHARED((N_PAD,), jnp.float32),
        ],
    )
    return f(acc_p, hc2, ntt, a_c, svec)


def _cls2_body(xcls_hbm, u3_hbm, hc_hbm, sumC_hbm,
               xeh_hbm,
               sc0_v, sc1_v, hc_v, hc1_v, u3f_v, gm_v, xc_v, zz_v,
               xeh_sp):
    cid = lax.axis_index("c")
    sid = lax.axis_index("s")
    wid = sid * NC + cid
    pltpu.sync_copy(sumC_hbm.at[pl.ds(0, N_PAD)], sc0_v)
    pltpu.sync_copy(sumC_hbm.at[pl.ds(N_PAD, N_PAD)], sc1_v)
    pltpu.sync_copy(hc_hbm.at[pl.ds(wid * 256, 256)], hc1_v)
    pltpu.sync_copy(u3_hbm.at[pl.ds(wid * 256, 256)], u3f_v)
    pltpu.sync_copy(xcls_hbm.at[pl.ds(wid * CW, CW)], xc_v)

    def _hcb(g, _):
        hc_v[g] = hc1_v[pl.ds(g * L, L)]
        return 0
    lax.fori_loop(0, CWG, _hcb, 0)

    def _zr(r, _):
        for l in range(8):
            zz_v[r, pl.ds(l * L, L)] = jnp.zeros((L,), jnp.float32)
        return 0
    lax.fori_loop(0, 16, _zr, 0)

    def _zc(i, _):
        pltpu.sync_copy(zz_v, xeh_sp.at[pl.ds(sid * NSEG + i * 16, 16)])
        return 0
    lax.fori_loop(0, NSEG // 16, _zc, 0)
    plsc.subcore_barrier()

    def _grp(g, _):
        hcx = hc_v[g]
        sc = plsc.load_gather(sc0_v, [hcx]) + plsc.load_gather(sc1_v, [hcx])
        gm_v[g] = u3f_v[pl.ds(g * L, L)] / (sc + 1e-38)
        return 0
    lax.fori_loop(0, CWG, _grp, 0)

    def _scale(r, _):
        wb = plsc.load_gather(
            gm_v, [jnp.full((L,), r // L, jnp.int32),
                   jnp.full((L,), r % L, jnp.int32)])
        for l in range(8):
            xc_v[r, pl.ds(l * L, L)] = xc_v[r, pl.ds(l * L, L)] * wb
        return 0
    lax.fori_loop(0, CW, _scale, 0)

    def _scat(g, _):
        pltpu.sync_copy(xc_v.at[pl.ds(g * L, L)], xeh_sp.at[hc_v.at[g]],
                        add=True)
        return 0
    lax.fori_loop(0, CWG, _scat, 0)
    plsc.subcore_barrier()
    pltpu.sync_copy(xeh_sp.at[pl.ds(sid * NSEG, NSEG)],
                    xeh_hbm.at[pl.ds(cid * N_PAD + sid * NSEG, NSEG)])


def _cls2_call(xcls, u3, hc2, sumC_p):
    f = pl.kernel(
        _cls2_body,
        out_type=[
            jax.ShapeDtypeStruct((NC * N_PAD, H), jnp.float32),
        ],
        mesh=plsc.VectorSubcoreMesh(core_axis_name="c", subcore_axis_name="s"),
        compiler_params=pltpu.CompilerParams(needs_layout_passes=False),
        scratch_types=[
            pltpu.VMEM((N_PAD,), jnp.float32),
            pltpu.VMEM((N_PAD,), jnp.float32),
            pltpu.VMEM((L, L), jnp.int32),
            pltpu.VMEM((256,), jnp.int32),
            pltpu.VMEM((256,), jnp.float32),
            pltpu.VMEM((L, L), jnp.float32),
            pltpu.VMEM((CW, H), jnp.float32),
            pltpu.VMEM((16, H), jnp.float32),
            pltpu.VMEM_SHARED((N_PAD, H), jnp.float32),
        ],
    )
    return f(xcls, u3, hc2, sumC_p)[0]


def _gate_body(h_ref, e0_ref, e1_ref, w_ref, b_ref, o_ref):
    h = h_ref[...]
    g = jax.nn.sigmoid(jax.lax.dot(h, w_ref[...], preferred_element_type=jnp.float32)
                       + b_ref[...][None, :])
    o_ref[...] = g * (e0_ref[...] + e1_ref[...]) + (1.0 - g) * h


def _gate_call(x_r_h, xeh0, xeh1, hw_W, hw_b):
    return pl.pallas_call(
        _gate_body,
        grid=(N // NB,),
        in_specs=[
            pl.BlockSpec((NB, H), lambda i: (i, 0)),
            pl.BlockSpec((NB, H), lambda i: (i, 0)),
            pl.BlockSpec((NB, H), lambda i: (i, 0)),
            pl.BlockSpec((H, H), lambda i: (0, 0)),
            pl.BlockSpec((H,), lambda i: (0,)),
        ],
        out_specs=pl.BlockSpec((NB, H), lambda i: (i, 0)),
        out_shape=jax.ShapeDtypeStruct((N, H), jnp.float32),
    )(x_r_h, xeh0, xeh1, hw_W, hw_b)


def kernel(x_e, edge_index, rel, triple_num, r_emb, class_index, head_class,
           a_h1, a_h2, a_h3, a_h4, a_t1, a_t2, a_t3, a_r1, a_r2, a_c,
           W_h, W_t, hw_W, hw_b):
    Ah = jnp.stack([a_h1, a_h2, a_h3, a_h4], axis=1)
    At = jnp.stack([a_t1, a_t2, a_t3], axis=1)
    Ar = jnp.stack([a_r1, a_r2], axis=1)
    x_r_h, x_r_t, ntab = _nodes_call(x_e, W_h, W_t, Ah, At)
    rtab, svec, ntt = _rel_call(ntab, r_emb, Ar, a_t2, a_t3, a_c)
    S1, S2, S3 = svec[0, 0], svec[1, 0], svec[2, 0]
    q1, q2, q3, q4 = ntt[0], ntt[1], ntt[2], ntt[3]
    t1, t2, t3, nrm = ntt[4], ntt[5], ntt[6], ntt[7]
    r1, r2 = rtab[0], rtab[1]
    eh, et = edge_index[0], edge_index[1]
    tn, ci, hc = triple_num, class_index, head_class
    EPS = 1e-38

    pad_i = jnp.zeros((E_PAD - E,), jnp.int32)
    eh2 = jnp.concatenate([eh, pad_i]).reshape(ER, 128)
    et2 = jnp.concatenate([et, pad_i]).reshape(ER, 128)
    rel2 = jnp.concatenate([rel, pad_i]).reshape(ER, 128)
    tn2 = jnp.concatenate([tn, jnp.full((E_PAD - E,), N_PAD - 1, jnp.int32)]
                          ).reshape(ER, 128)
    ci2 = jnp.concatenate([ci, jnp.full((E_PAD - E,), C_PAD - 1, jnp.int32)]
                          ).reshape(ER, 128)
    hcp = jnp.concatenate([hc, jnp.full((C_PAD - C,), N_PAD - 1, jnp.int32)])
    hc3 = jnp.pad(hcp.reshape(NW, CW), ((0, 0), (0, 256 - CW)),
                  constant_values=N_PAD - 1).reshape(NW * 256)
    ntt_f = ntt.reshape(8 * N)
    rtab_f = rtab.reshape(2 * R)
    svec_f = svec.reshape(8 * L)

    u1_2, sumA_p = _edge1_call(eh2, et2, rel2, tn2, ntt_f, rtab_f, svec_f)
    v2, sumB_p = _edge2_call(eh2, et2, rel2, tn2, ci2, u1_2, ntt_f, rtab_f,
                             svec_f, sumA_p)
    xrt2 = jnp.concatenate([x_r_t, x_r_t], axis=0)
    acc_p = _edge3_call(et2, ci2, v2, xrt2, sumB_p)
    xcls, u3, sumC_p = _cls1_call(acc_p, hc3, ntt_f, a_c, svec_f)
    xeh_p = _cls2_call(xcls, u3, hc3, sumC_p)

    return _gate_call(x_r_h, xeh_p[:N], xeh_p[N_PAD:N_PAD + N], hw_W, hw_b)


# speedup vs baseline: 1.1750x; 1.1750x over previous
import functools
import jax
import jax.numpy as jnp
from jax import lax
from jax.experimental import pallas as pl
from jax.experimental.pallas import tpu as pltpu
from jax.experimental.pallas import tpu_sc as plsc

N = 10000
E = 320000
H = 128
R = 1000
C = 5000

NB = 1000
N_PAD = 10240
C_PAD = 5120
E_PAD = 327680

NC = 2
NS = 16
L = 16
NW = NC * NS
EW = E_PAD // NW
ER = E_PAD // 128
WR = EW // 128
CHR = 16
NSEG = N_PAD // NS
CSEG = C_PAD // NS

_ZIDX = functools.partial(jnp.full, (L,), dtype=jnp.int32)


def _bcast0(ref):
    return plsc.load_gather(ref, [jnp.zeros((L,), jnp.int32)])


def _lk(x):
    return jnp.where(x >= 0, x, 0.01 * x)


def _nodes_body(x_ref, wh_ref, wt_ref, ah_ref, at_ref, xh_ref, xt_ref, nt_ref):
    x = x_ref[...]
    h = jax.nn.relu(jax.lax.dot(x, wh_ref[...], preferred_element_type=jnp.float32))
    t = jax.nn.relu(jax.lax.dot(x, wt_ref[...], preferred_element_type=jnp.float32))
    xh_ref[...] = h
    xt_ref[...] = t
    sh = jax.lax.dot_general(ah_ref[...], h, (((0,), (1,)), ((), ())),
                             preferred_element_type=jnp.float32)
    st = jax.lax.dot_general(at_ref[...], t, (((0,), (1,)), ((), ())),
                             preferred_element_type=jnp.float32)
    nrm = jnp.sqrt(jnp.sum(t * t, axis=1))[None, :]
    nt_ref[...] = jnp.concatenate([sh, st, nrm], axis=0).T


def _nodes_call(x_e, W_h, W_t, Ah, At):
    return pl.pallas_call(
        _nodes_body,
        grid=(N // NB,),
        in_specs=[
            pl.BlockSpec((NB, H), lambda i: (i, 0)),
            pl.BlockSpec((H, H), lambda i: (0, 0)),
            pl.BlockSpec((H, H), lambda i: (0, 0)),
            pl.BlockSpec((H, 4), lambda i: (0, 0)),
            pl.BlockSpec((H, 3), lambda i: (0, 0)),
        ],
        out_specs=[
            pl.BlockSpec((NB, H), lambda i: (i, 0)),
            pl.BlockSpec((NB, H), lambda i: (i, 0)),
            pl.BlockSpec((NB, 8), lambda i: (i, 0)),
        ],
        out_shape=[
            jax.ShapeDtypeStruct((N, H), jnp.float32),
            jax.ShapeDtypeStruct((N, H), jnp.float32),
            jax.ShapeDtypeStruct((N, 8), jnp.float32),
        ],
    )(x_e, W_h, W_t, Ah, At)


def _rel_body(nt_ref, re_ref, ar_ref, at2_ref, at3_ref, ac_ref, rt_ref, sv_ref,
              ntt_ref):
    rt = jax.lax.dot_general(ar_ref[...], re_ref[...], (((0,), (1,)), ((), ())),
                             preferred_element_type=jnp.float32)
    rt_ref[...] = rt
    nt = nt_ref[...].T
    ntt_ref[...] = nt
    mq1, mq2, mq3, mq4 = (jnp.max(nt[0]), jnp.max(nt[1]), jnp.max(nt[2]),
                          jnp.max(nt[3]))
    mt1 = jnp.max(nt[4])
    mr1 = jnp.max(rt[0])
    mr2 = jnp.max(rt[1])
    nt2 = jnp.sqrt(jnp.sum(at2_ref[...] ** 2))
    nt3 = jnp.sqrt(jnp.sum(at3_ref[...] ** 2))
    nac = jnp.sqrt(jnp.sum(ac_ref[...] ** 2))
    s1 = _lk((mq1 + mt1) / 2.0 + mr1)
    s2 = _lk(nt2 + mq2 + (mr2 + (mq3 + nt3) / 2.0) / 2.0)
    s3 = _lk(nac + mq4)
    sv = jnp.concatenate([jnp.stack([s1, s2, s3]), jnp.zeros((5,), jnp.float32)])
    sv_ref[...] = jnp.broadcast_to(sv[:, None], (8, 16))


def _rel_call(ntab, r_emb, Ar, a_t2, a_t3, a_c):
    return pl.pallas_call(
        _rel_body,
        in_specs=[
            pl.BlockSpec((N, 8), lambda: (0, 0)),
            pl.BlockSpec((R, H), lambda: (0, 0)),
            pl.BlockSpec((H, 2), lambda: (0, 0)),
            pl.BlockSpec((H,), lambda: (0,)),
            pl.BlockSpec((H,), lambda: (0,)),
            pl.BlockSpec((H,), lambda: (0,)),
        ],
        out_specs=[
            pl.BlockSpec((2, R), lambda: (0, 0)),
            pl.BlockSpec((8, 16), lambda: (0, 0)),
            pl.BlockSpec((8, N), lambda: (0, 0)),
        ],
        out_shape=[
            jax.ShapeDtypeStruct((2, R), jnp.float32),
            jax.ShapeDtypeStruct((8, 16), jnp.float32),
            jax.ShapeDtypeStruct((8, N), jnp.float32),
        ],
    )(ntab, r_emb, Ar, a_t2, a_t3, a_c)


def _edge1_body(eh_hbm, et_hbm, rel_hbm, tn_hbm, ntt_hbm, rtab_hbm, svec_hbm,
                u1_hbm, sumA_hbm,
                q1_v, t1_v, r1_v, sv_v, eh_v, et_v, rel_v, tn_v, u1_v, zz_v,
                sumA_sp):
    cid = lax.axis_index("c")
    sid = lax.axis_index("s")
    wid = sid * NC + cid
    pltpu.sync_copy(ntt_hbm.at[pl.ds(0, N)], q1_v)
    pltpu.sync_copy(ntt_hbm.at[pl.ds(4 * N, N)], t1_v)
    pltpu.sync_copy(rtab_hbm.at[pl.ds(0, R)], r1_v)
    pltpu.sync_copy(svec_hbm.at[pl.ds(0, L)], sv_v)

    def _zf(i, _):
        zz_v[pl.ds(i * L, L)] = jnp.zeros((L,), jnp.float32)
        return 0
    lax.fori_loop(0, NSEG // L, _zf, 0)
    pltpu.sync_copy(zz_v, sumA_sp.at[pl.ds(sid * NSEG, NSEG)])
    plsc.subcore_barrier()
    S1 = sv_v[...]
    base_row = wid * WR

    def _chunk(ch, _):
        r0 = base_row + ch * CHR
        pltpu.sync_copy(eh_hbm.at[pl.ds(r0, CHR)], eh_v)
        pltpu.sync_copy(et_hbm.at[pl.ds(r0, CHR)], et_v)
        pltpu.sync_copy(rel_hbm.at[pl.ds(r0, CHR)], rel_v)
        pltpu.sync_copy(tn_hbm.at[pl.ds(r0, CHR)], tn_v)

        def _row(r, _):
            for l in range(8):
                ehx = eh_v[r, pl.ds(l * L, L)]
                etx = et_v[r, pl.ds(l * L, L)]
                rlx = rel_v[r, pl.ds(l * L, L)]
                e1 = (plsc.load_gather(q1_v, [ehx])
                      + plsc.load_gather(t1_v, [etx])) * 0.5 \
                     + plsc.load_gather(r1_v, [rlx])
                z = jnp.where(e1 >= 0, e1, 0.01 * e1)
                u1_v[r, pl.ds(l * L, L)] = jnp.exp(z - S1)
            return 0
        lax.fori_loop(0, CHR, _row, 0)
        pltpu.sync_copy(u1_v, u1_hbm.at[pl.ds(r0, CHR)])

        def _srow(r, _):
            pltpu.sync_copy(u1_v.at[r], sumA_sp.at[tn_v.at[r]], add=True)
            return 0
        lax.fori_loop(0, CHR, _srow, 0)
        return 0
    lax.fori_loop(0, EW // (CHR * 128), _chunk, 0)
    plsc.subcore_barrier()
    pltpu.sync_copy(sumA_sp.at[pl.ds(sid * NSEG, NSEG)],
                    sumA_hbm.at[pl.ds(cid * N_PAD + sid * NSEG, NSEG)])


def _edge1_call(eh2, et2, rel2, tn2, ntt, rtab, svec):
    f = pl.kernel(
        _edge1_body,
        out_type=[
            jax.ShapeDtypeStruct((ER, 128), jnp.float32),
            jax.ShapeDtypeStruct((NC * N_PAD,), jnp.float32),
        ],
        mesh=plsc.VectorSubcoreMesh(core_axis_name="c", subcore_axis_name="s"),
        compiler_params=pltpu.CompilerParams(needs_layout_passes=False),
        scratch_types=[
            pltpu.VMEM((N,), jnp.float32),
            pltpu.VMEM((N,), jnp.float32),
            pltpu.VMEM((R,), jnp.float32),
            pltpu.VMEM((L,), jnp.float32),
            pltpu.VMEM((CHR, 128), jnp.int32),
            pltpu.VMEM((CHR, 128), jnp.int32),
            pltpu.VMEM((CHR, 128), jnp.int32),
            pltpu.VMEM((CHR, 128), jnp.int32),
            pltpu.VMEM((CHR, 128), jnp.float32),
            pltpu.VMEM((NSEG,), jnp.float32),
            pltpu.VMEM_SHARED((N_PAD,), jnp.float32),
        ],
    )
    return f(eh2, et2, rel2, tn2, ntt, rtab, svec)


def _edge2_body(eh_hbm, et_hbm, rel_hbm, tn_hbm, ci_hbm, u1_hbm,
                ntt_hbm, rtab_hbm, svec_hbm, sA_hbm,
                v_hbm, sumB_hbm,
                q2_v, q3_v, t2_v, t3_v, nrm_v, r2_v, sv_v, sa0_v, sa1_v,
                eh_v, et_v, rel_v, tn_v, ci_v, u1_v, u2_v, v_v, zz_v,
                sumB_sp):
    cid = lax.axis_index("c")
    sid = lax.axis_index("s")
    wid = sid * NC + cid
    pltpu.sync_copy(ntt_hbm.at[pl.ds(1 * N, N)], q2_v)
    pltpu.sync_copy(ntt_hbm.at[pl.ds(2 * N, N)], q3_v)
    pltpu.sync_copy(ntt_hbm.at[pl.ds(5 * N, N)], t2_v)
    pltpu.sync_copy(ntt_hbm.at[pl.ds(6 * N, N)], t3_v)
    pltpu.sync_copy(ntt_hbm.at[pl.ds(7 * N, N)], nrm_v)
    pltpu.sync_copy(rtab_hbm.at[pl.ds(R, R)], r2_v)
    pltpu.sync_copy(svec_hbm.at[pl.ds(L, L)], sv_v)
    pltpu.sync_copy(sA_hbm.at[pl.ds(0, N_PAD)], sa0_v)
    pltpu.sync_copy(sA_hbm.at[pl.ds(N_PAD, N_PAD)], sa1_v)

    def _zf(i, _):
        zz_v[pl.ds(i * L, L)] = jnp.zeros((L,), jnp.float32)
        return 0
    lax.fori_loop(0, NSEG // L, _zf, 0)

    @pl.when(sid < 8)
    def _zero_sumb():
        pltpu.sync_copy(zz_v, sumB_sp.at[pl.ds(sid * NSEG, NSEG)])
    plsc.subcore_barrier()

    S2 = sv_v[...]
    base_row = wid * WR

    def _chunk(ch, _):
        r0 = base_row + ch * CHR
        pltpu.sync_copy(eh_hbm.at[pl.ds(r0, CHR)], eh_v)
        pltpu.sync_copy(et_hbm.at[pl.ds(r0, CHR)], et_v)
        pltpu.sync_copy(rel_hbm.at[pl.ds(r0, CHR)], rel_v)
        pltpu.sync_copy(tn_hbm.at[pl.ds(r0, CHR)], tn_v)
        pltpu.sync_copy(ci_hbm.at[pl.ds(r0, CHR)], ci_v)
        pltpu.sync_copy(u1_hbm.at[pl.ds(r0, CHR)], u1_v)

        def _row(r, _):
            for l in range(8):
                ehx = eh_v[r, pl.ds(l * L, L)]
                etx = et_v[r, pl.ds(l * L, L)]
                rlx = rel_v[r, pl.ds(l * L, L)]
                tnx = tn_v[r, pl.ds(l * L, L)]
                u1x = u1_v[r, pl.ds(l * L, L)]
                sa = (plsc.load_gather(sa0_v, [tnx])
                      + plsc.load_gather(sa1_v, [tnx]))
                alpha = u1x / (sa + 1e-38)
                nr = plsc.load_gather(nrm_v, [etx])
                s = alpha / jnp.maximum(nr * alpha, 1e-12)
                e = (s * plsc.load_gather(t2_v, [etx])
                     + plsc.load_gather(q2_v, [ehx])
                     + (plsc.load_gather(r2_v, [rlx])
                        + (plsc.load_gather(q3_v, [ehx])
                           + s * plsc.load_gather(t3_v, [etx])) * 0.5) * 0.5)
                z = jnp.where(e >= 0, e, 0.01 * e)
                u2 = jnp.exp(z - S2)
                u2_v[r, pl.ds(l * L, L)] = u2
                v_v[r, pl.ds(l * L, L)] = s * u2
            return 0
        lax.fori_loop(0, CHR, _row, 0)
        pltpu.sync_copy(v_v, v_hbm.at[pl.ds(r0, CHR)])

        def _srow(r, _):
            pltpu.sync_copy(u2_v.at[r], sumB_sp.at[ci_v.at[r]], add=True)
            return 0
        lax.fori_loop(0, CHR, _srow, 0)
        return 0
    lax.fori_loop(0, EW // (CHR * 128), _chunk, 0)
    plsc.subcore_barrier()

    @pl.when(sid < 8)
    def _write_sumb():
        pltpu.sync_copy(sumB_sp.at[pl.ds(sid * NSEG, NSEG)],
                        sumB_hbm.at[pl.ds(cid * C_PAD + sid * NSEG, NSEG)])


def _edge2_call(eh2, et2, rel2, tn2, ci2, u1_2, ntt, rtab, svec, sumA_p):
    f = pl.kernel(
        _edge2_body,
        out_type=[
            jax.ShapeDtypeStruct((ER, 128), jnp.float32),
            jax.ShapeDtypeStruct((NC * C_PAD,), jnp.float32),
        ],
        mesh=plsc.VectorSubcoreMesh(core_axis_name="c", subcore_axis_name="s"),
        compiler_params=pltpu.CompilerParams(needs_layout_passes=False),
        scratch_types=[
            pltpu.VMEM((N,), jnp.float32),
            pltpu.VMEM((N,), jnp.float32),
            pltpu.VMEM((N,), jnp.float32),
            pltpu.VMEM((N,), jnp.float32),
            pltpu.VMEM((N,), jnp.float32),
            pltpu.VMEM((R,), jnp.float32),
            pltpu.VMEM((L,), jnp.float32),
            pltpu.VMEM((N_PAD,), jnp.float32),
            pltpu.VMEM((N_PAD,), jnp.float32),
            pltpu.VMEM((CHR, 128), jnp.int32),
            pltpu.VMEM((CHR, 128), jnp.int32),
            pltpu.VMEM((CHR, 128), jnp.int32),
            pltpu.VMEM((CHR, 128), jnp.int32),
            pltpu.VMEM((CHR, 128), jnp.int32),
            pltpu.VMEM((CHR, 128), jnp.float32),
            pltpu.VMEM((CHR, 128), jnp.float32),
            pltpu.VMEM((CHR, 128), jnp.float32),
            pltpu.VMEM((NSEG,), jnp.float32),
            pltpu.VMEM_SHARED((C_PAD,), jnp.float32),
        ],
    )
    return f(eh2, et2, rel2, tn2, ci2, u1_2, ntt, rtab, svec, sumA_p)


def _edge3_body(et_hbm, ci_hbm, v_hbm, xrt_hbm, sumB_hbm,
                acc_hbm,
                sb0_v, sb1_v, et_v, ci_v, v_v, w_v, rows_v, rows1_v, rows2_v,
                rows3_v, zz_v, acc_sp, sem, ssem):
    cid = lax.axis_index("c")
    sid = lax.axis_index("s")
    wid = sid * NC + cid
    pltpu.sync_copy(sumB_hbm.at[pl.ds(0, C_PAD)], sb0_v)
    pltpu.sync_copy(sumB_hbm.at[pl.ds(C_PAD, C_PAD)], sb1_v)

    def _zr(r, _):
        for l in range(8):
            zz_v[r, pl.ds(l * L, L)] = jnp.zeros((L,), jnp.float32)
        return 0
    lax.fori_loop(0, 16, _zr, 0)

    def _zc(i, _):
        pltpu.sync_copy(zz_v, acc_sp.at[pl.ds(sid * CSEG + i * 16, 16)])
        return 0
    lax.fori_loop(0, CSEG // 16, _zc, 0)
    plsc.subcore_barrier()

    base_row = wid * WR

    def _chunk(ch, _):
        r0 = base_row + ch * CHR
        pltpu.sync_copy(et_hbm.at[pl.ds(r0, CHR)], et_v)
        pltpu.sync_copy(ci_hbm.at[pl.ds(r0, CHR)], ci_v)
        pltpu.sync_copy(v_hbm.at[pl.ds(r0, CHR)], v_v)

        def _wrow(r, _):
            for l in range(8):
                cix = ci_v[r, pl.ds(l * L, L)]
                sb = (plsc.load_gather(sb0_v, [cix])
                      + plsc.load_gather(sb1_v, [cix]))
                w_v[r, pl.ds(l * L, L)] = v_v[r, pl.ds(l * L, L)] / (sb + 1e-38)
            return 0
        lax.fori_loop(0, CHR, _wrow, 0)

        def _scale_row(rows_b, r):
            def _sg(g, _):
                for i in range(L):
                    e = g * L + i
                    wb = plsc.load_gather(
                        w_v, [jnp.full((L,), r, jnp.int32),
                              jnp.full((L,), e, jnp.int32)])
                    for l in range(8):
                        rows_b[e, pl.ds(l * L, L)] = (
                            rows_b[e, pl.ds(l * L, L)] * wb)
                return 0
            lax.fori_loop(0, 128 // L, _sg, 0)

        bufs = (rows_v, rows1_v, rows2_v, rows3_v)

        def _drain(sm):
            pltpu.make_async_copy(xrt_hbm.at[pl.ds(0, 128)], rows_v, sm).wait()

        pltpu.async_copy(xrt_hbm.at[et_v.at[0]], bufs[0], sem)
        pltpu.async_copy(xrt_hbm.at[et_v.at[1]], bufs[1], sem)

        def _quad(q, _):
            for i in range(4):
                r = 4 * q + i

                @pl.when(r >= 2)
                def _drain_scat():
                    _drain(ssem)

                @pl.when(r + 2 < CHR)
                def _fire_gather():
                    pltpu.async_copy(xrt_hbm.at[et_v.at[r + 2]],
                                     bufs[(i + 2) % 4], sem)
                _drain(sem)
                _scale_row(bufs[i], r)
                pltpu.async_copy(bufs[i], acc_sp.at[ci_v.at[r]], ssem,
                                 add=True)
            return 0
        lax.fori_loop(0, CHR // 4, _quad, 0)
        _drain(ssem)
        _drain(ssem)
        return 0
    lax.fori_loop(0, EW // (CHR * 128), _chunk, 0)
    plsc.subcore_barrier()
    pltpu.sync_copy(acc_sp.at[pl.ds(sid * CSEG, CSEG)],
                    acc_hbm.at[pl.ds(cid * C_PAD + sid * CSEG, CSEG)])


def _edge3_call(et2, ci2, v2, x_r_t, sumB_p):
    f = pl.kernel(
        _edge3_body,
        out_type=[
            jax.ShapeDtypeStruct((NC * C_PAD, H), jnp.float32),
        ],
        mesh=plsc.VectorSubcoreMesh(core_axis_name="c", subcore_axis_name="s"),
        compiler_params=pltpu.CompilerParams(needs_layout_passes=False),
        scratch_types=[
            pltpu.VMEM((C_PAD,), jnp.float32),
            pltpu.VMEM((C_PAD,), jnp.float32),
            pltpu.VMEM((CHR, 128), jnp.int32),
            pltpu.VMEM((CHR, 128), jnp.int32),
            pltpu.VMEM((CHR, 128), jnp.float32),
            pltpu.VMEM((CHR, 128), jnp.float32),
            pltpu.VMEM((128, H), jnp.float32),
            pltpu.VMEM((128, H), jnp.float32),
            pltpu.VMEM((128, H), jnp.float32),
            pltpu.VMEM((128, H), jnp.float32),
            pltpu.VMEM((16, H), jnp.float32),
            pltpu.VMEM_SHARED((C_PAD, H), jnp.float32),
            pltpu.SemaphoreType.DMA,
            pltpu.SemaphoreType.DMA,
        ],
    )
    return f(et2, ci2, v2, x_r_t, sumB_p)[0]


CW = C_PAD // NW
CWG = CW // L


def _cls1_body(acc_hbm, hc_hbm, ntt_hbm, ac_hbm, svec_hbm,
               xcls_hbm, u3_hbm, sumC_hbm,
               q4_v, ac_v, sv_v, hc_v, hc1_v, u3_v, u3f_v, ec_v, xc_v, xc1_v,
               zz_v, sumC_sp):
    cid = lax.axis_index("c")
    sid = lax.axis_index("s")
    wid = sid * NC + cid
    pltpu.sync_copy(ntt_hbm.at[pl.ds(3 * N, N)], q4_v)
    pltpu.sync_copy(ac_hbm, ac_v)
    pltpu.sync_copy(svec_hbm.at[pl.ds(2 * L, L)], sv_v)
    pltpu.sync_copy(hc_hbm.at[pl.ds(wid * 256, 256)], hc1_v)

    def _hcb(g, _):
        hc_v[g] = hc1_v[pl.ds(g * L, L)]
        return 0
    lax.fori_loop(0, CWG, _hcb, 0)

    def _zf(i, _):
        zz_v[pl.ds(i * L, L)] = jnp.zeros((L,), jnp.float32)
        return 0
    lax.fori_loop(0, NSEG // L, _zf, 0)
    pltpu.sync_copy(zz_v, sumC_sp.at[pl.ds(sid * NSEG, NSEG)])
    plsc.subcore_barrier()

    pltpu.sync_copy(acc_hbm.at[pl.ds(wid * CW, CW)], xc_v)
    pltpu.sync_copy(acc_hbm.at[pl.ds(C_PAD + wid * CW, CW)], xc1_v)

    def _addr(r, _):
        for l in range(8):
            xc_v[r, pl.ds(l * L, L)] = (xc_v[r, pl.ds(l * L, L)]
                                        + xc1_v[r, pl.ds(l * L, L)])
        return 0
    lax.fori_loop(0, CW, _addr, 0)
    pltpu.sync_copy(xc_v, xcls_hbm.at[pl.ds(wid * CW, CW)])

    def _dotg(g, _):
        rows16 = g * L + lax.broadcasted_iota(jnp.int32, (L,), 0)

        def _j(j, acc):
            colv = plsc.load_gather(xc_v, [rows16, jnp.full((L,), j, jnp.int32)])
            acb = plsc.load_gather(ac_v, [jnp.full((L,), j, jnp.int32)])
            return acc + colv * acb
        ec_v[g] = lax.fori_loop(0, H, _j, jnp.zeros((L,), jnp.float32))
        return 0
    lax.fori_loop(0, CWG, _dotg, 0)

    S3 = sv_v[...]

    def _grp(g, _):
        hcx = hc_v[g]
        q4g = plsc.load_gather(q4_v, [jnp.minimum(hcx, N - 1)])
        e_c = ec_v[g] + q4g
        z = jnp.where(e_c >= 0, e_c, 0.01 * e_c)
        u3g = jnp.where(hcx < N, jnp.exp(z - S3), 0.0)
        u3_v[g] = u3g
        u3f_v[pl.ds(g * L, L)] = u3g
        pltpu.sync_copy(u3_v.at[g], sumC_sp.at[hc_v.at[g]], add=True)
        return 0
    lax.fori_loop(0, CWG, _grp, 0)
    pltpu.sync_copy(u3f_v, u3_hbm.at[pl.ds(wid * 256, 256)])
    plsc.subcore_barrier()
    pltpu.sync_copy(sumC_sp.at[pl.ds(sid * NSEG, NSEG)],
                    sumC_hbm.at[pl.ds(cid * N_PAD + sid * NSEG, NSEG)])


def _cls1_call(acc_p, hc2, ntt, a_c, svec):
    f = pl.kernel(
        _cls1_body,
        out_type=[
            jax.ShapeDtypeStruct((C_PAD, H), jnp.float32),
            jax.ShapeDtypeStruct((NW * 256,), jnp.float32),
            jax.ShapeDtypeStruct((NC * N_PAD,), jnp.float32),
        ],
        mesh=plsc.VectorSubcoreMesh(core_axis_name="c", subcore_axis_name="s"),
        compiler_params=pltpu.CompilerParams(needs_layout_passes=False),
        scratch_types=[
            pltpu.VMEM((N,), jnp.float32),
            pltpu.VMEM((H,), jnp.float32),
            pltpu.VMEM((L,), jnp.float32),
            pltpu.VMEM((L, L), jnp.int32),
            pltpu.VMEM((256,), jnp.int32),
            pltpu.VMEM((L, L), jnp.float32),
            pltpu.VMEM((256,), jnp.float32),
            pltpu.VMEM((L, L), jnp.float32),
            pltpu.VMEM((CW, H), jnp.float32),
            pltpu.VMEM((CW, H), jnp.float32),
            pltpu.VMEM((NSEG,), jnp.float32),
            pltpu.VMEM_SHARED((N_PAD,), jnp.float32),
        ],
    )
    return f(acc_p, hc2, ntt, a_c, svec)


def _cls2_body(xcls_hbm, u3_hbm, hc_hbm, sumC_hbm,
               xeh_hbm,
               sc0_v, sc1_v, hc_v, hc1_v, u3f_v, gm_v, xc_v, zz_v,
               xeh_sp):
    cid = lax.axis_index("c")
    sid = lax.axis_index("s")
    wid = sid * NC + cid
    pltpu.sync_copy(sumC_hbm.at[pl.ds(0, N_PAD)], sc0_v)
    pltpu.sync_copy(sumC_hbm.at[pl.ds(N_PAD, N_PAD)], sc1_v)
    pltpu.sync_copy(hc_hbm.at[pl.ds(wid * 256, 256)], hc1_v)
    pltpu.sync_copy(u3_hbm.at[pl.ds(wid * 256, 256)], u3f_v)
    pltpu.sync_copy(xcls_hbm.at[pl.ds(wid * CW, CW)], xc_v)

    def _hcb(g, _):
        hc_v[g] = hc1_v[pl.ds(g * L, L)]
        return 0
    lax.fori_loop(0, CWG, _hcb, 0)

    def _zr(r, _):
        for l in range(8):
            zz_v[r, pl.ds(l * L, L)] = jnp.zeros((L,), jnp.float32)
        return 0
    lax.fori_loop(0, 16, _zr, 0)

    def _zc(i, _):
        pltpu.sync_copy(zz_v, xeh_sp.at[pl.ds(sid * NSEG + i * 16, 16)])
        return 0
    lax.fori_loop(0, NSEG // 16, _zc, 0)
    plsc.subcore_barrier()

    def _grp(g, _):
        hcx = hc_v[g]
        sc = plsc.load_gather(sc0_v, [hcx]) + plsc.load_gather(sc1_v, [hcx])
        gm_v[g] = u3f_v[pl.ds(g * L, L)] / (sc + 1e-38)
        return 0
    lax.fori_loop(0, CWG, _grp, 0)

    def _scale(r, _):
        wb = plsc.load_gather(
            gm_v, [jnp.full((L,), r // L, jnp.int32),
                   jnp.full((L,), r % L, jnp.int32)])
        for l in range(8):
            xc_v[r, pl.ds(l * L, L)] = xc_v[r, pl.ds(l * L, L)] * wb
        return 0
    lax.fori_loop(0, CW, _scale, 0)

    def _scat(g, _):
        pltpu.sync_copy(xc_v.at[pl.ds(g * L, L)], xeh_sp.at[hc_v.at[g]],
                        add=True)
        return 0
    lax.fori_loop(0, CWG, _scat, 0)
    plsc.subcore_barrier()
    pltpu.sync_copy(xeh_sp.at[pl.ds(sid * NSEG, NSEG)],
                    xeh_hbm.at[pl.ds(cid * N_PAD + sid * NSEG, NSEG)])


def _cls2_call(xcls, u3, hc2, sumC_p):
    f = pl.kernel(
        _cls2_body,
        out_type=[
            jax.ShapeDtypeStruct((NC * N_PAD, H), jnp.float32),
        ],
        mesh=plsc.VectorSubcoreMesh(core_axis_name="c", subcore_axis_name="s"),
        compiler_params=pltpu.CompilerParams(needs_layout_passes=False),
        scratch_types=[
            pltpu.VMEM((N_PAD,), jnp.float32),
            pltpu.VMEM((N_PAD,), jnp.float32),
            pltpu.VMEM((L, L), jnp.int32),
            pltpu.VMEM((256,), jnp.int32),
            pltpu.VMEM((256,), jnp.float32),
            pltpu.VMEM((L, L), jnp.float32),
            pltpu.VMEM((CW, H), jnp.float32),
            pltpu.VMEM((16, H), jnp.float32),
            pltpu.VMEM_SHARED((N_PAD, H), jnp.float32),
        ],
    )
    return f(xcls, u3, hc2, sumC_p)[0]


def _gate_body(h_ref, e0_ref, e1_ref, w_ref, b_ref, o_ref):
    h = h_ref[...]
    g = jax.nn.sigmoid(jax.lax.dot(h, w_ref[...], preferred_element_type=jnp.float32)
                       + b_ref[...][None, :])
    o_ref[...] = g * (e0_ref[...] + e1_ref[...]) + (1.0 - g) * h


def _gate_call(x_r_h, xeh0, xeh1, hw_W, hw_b):
    return pl.pallas_call(
        _gate_body,
        grid=(N // NB,),
        in_specs=[
            pl.BlockSpec((NB, H), lambda i: (i, 0)),
            pl.BlockSpec((NB, H), lambda i: (i, 0)),
            pl.BlockSpec((NB, H), lambda i: (i, 0)),
            pl.BlockSpec((H, H), lambda i: (0, 0)),
            pl.BlockSpec((H,), lambda i: (0,)),
        ],
        out_specs=pl.BlockSpec((NB, H), lambda i: (i, 0)),
        out_shape=jax.ShapeDtypeStruct((N, H), jnp.float32),
    )(x_r_h, xeh0, xeh1, hw_W, hw_b)


def kernel(x_e, edge_index, rel, triple_num, r_emb, class_index, head_class,
           a_h1, a_h2, a_h3, a_h4, a_t1, a_t2, a_t3, a_r1, a_r2, a_c,
           W_h, W_t, hw_W, hw_b):
    Ah = jnp.stack([a_h1, a_h2, a_h3, a_h4], axis=1)
    At = jnp.stack([a_t1, a_t2, a_t3], axis=1)
    Ar = jnp.stack([a_r1, a_r2], axis=1)
    x_r_h, x_r_t, ntab = _nodes_call(x_e, W_h, W_t, Ah, At)
    rtab, svec, ntt = _rel_call(ntab, r_emb, Ar, a_t2, a_t3, a_c)
    S1, S2, S3 = svec[0, 0], svec[1, 0], svec[2, 0]
    q1, q2, q3, q4 = ntt[0], ntt[1], ntt[2], ntt[3]
    t1, t2, t3, nrm = ntt[4], ntt[5], ntt[6], ntt[7]
    r1, r2 = rtab[0], rtab[1]
    eh, et = edge_index[0], edge_index[1]
    tn, ci, hc = triple_num, class_index, head_class
    EPS = 1e-38

    pad_i = jnp.zeros((E_PAD - E,), jnp.int32)
    eh2 = jnp.concatenate([eh, pad_i]).reshape(ER, 128)
    et2 = jnp.concatenate([et, pad_i]).reshape(ER, 128)
    rel2 = jnp.concatenate([rel, pad_i]).reshape(ER, 128)
    tn2 = jnp.concatenate([tn, jnp.full((E_PAD - E,), N_PAD - 1, jnp.int32)]
                          ).reshape(ER, 128)
    ci2 = jnp.concatenate([ci, jnp.full((E_PAD - E,), C_PAD - 1, jnp.int32)]
                          ).reshape(ER, 128)
    hcp = jnp.concatenate([hc, jnp.full((C_PAD - C,), N_PAD - 1, jnp.int32)])
    hc3 = jnp.pad(hcp.reshape(NW, CW), ((0, 0), (0, 256 - CW)),
                  constant_values=N_PAD - 1).reshape(NW * 256)
    ntt_f = ntt.reshape(8 * N)
    rtab_f = rtab.reshape(2 * R)
    svec_f = svec.reshape(8 * L)

    u1_2, sumA_p = _edge1_call(eh2, et2, rel2, tn2, ntt_f, rtab_f, svec_f)
    v2, sumB_p = _edge2_call(eh2, et2, rel2, tn2, ci2, u1_2, ntt_f, rtab_f,
                             svec_f, sumA_p)
    acc_p = _edge3_call(et2, ci2, v2, x_r_t, sumB_p)
    xcls, u3, sumC_p = _cls1_call(acc_p, hc3, ntt_f, a_c, svec_f)
    xeh_p = _cls2_call(xcls, u3, hc3, sumC_p)

    return _gate_call(x_r_h, xeh_p[:N], xeh_p[N_PAD:N_PAD + N], hw_W, hw_b)


# speedup vs baseline: 1.2057x; 1.0261x over previous
import functools
import jax
import jax.numpy as jnp
from jax import lax
from jax.experimental import pallas as pl
from jax.experimental.pallas import tpu as pltpu
from jax.experimental.pallas import tpu_sc as plsc

N = 10000
E = 320000
H = 128
R = 1000
C = 5000

NB = 2000
N_PAD = 10240
C_PAD = 5120
E_PAD = 327680

NC = 2
NS = 16
L = 16
NW = NC * NS
EW = E_PAD // NW
ER = E_PAD // 128
WR = EW // 128
CHR = 16
FAST_CID = 0
FCH = 7
SCH = 3
NSEG = N_PAD // NS
CSEG = C_PAD // NS

_ZIDX = functools.partial(jnp.full, (L,), dtype=jnp.int32)


def _bcast0(ref):
    return plsc.load_gather(ref, [jnp.zeros((L,), jnp.int32)])


def _lk(x):
    return jnp.where(x >= 0, x, 0.01 * x)


def _nodes_body(x_ref, wh_ref, wt_ref, ah_ref, at_ref, xh_ref, xt_ref, nt_ref):
    x = x_ref[...]
    h = jax.nn.relu(jax.lax.dot(x, wh_ref[...], preferred_element_type=jnp.float32))
    t = jax.nn.relu(jax.lax.dot(x, wt_ref[...], preferred_element_type=jnp.float32))
    xh_ref[...] = h
    xt_ref[...] = t
    sh = jax.lax.dot_general(ah_ref[...], h, (((0,), (1,)), ((), ())),
                             preferred_element_type=jnp.float32)
    st = jax.lax.dot_general(at_ref[...], t, (((0,), (1,)), ((), ())),
                             preferred_element_type=jnp.float32)
    nrm = jnp.sqrt(jnp.sum(t * t, axis=1))[None, :]
    nt_ref[...] = jnp.concatenate([sh, st, nrm], axis=0).T


def _nodes_call(x_e, W_h, W_t, Ah, At):
    return pl.pallas_call(
        _nodes_body,
        grid=(N // NB,),
        in_specs=[
            pl.BlockSpec((NB, H), lambda i: (i, 0)),
            pl.BlockSpec((H, H), lambda i: (0, 0)),
            pl.BlockSpec((H, H), lambda i: (0, 0)),
            pl.BlockSpec((H, 4), lambda i: (0, 0)),
            pl.BlockSpec((H, 3), lambda i: (0, 0)),
        ],
        out_specs=[
            pl.BlockSpec((NB, H), lambda i: (i, 0)),
            pl.BlockSpec((NB, H), lambda i: (i, 0)),
            pl.BlockSpec((NB, 8), lambda i: (i, 0)),
        ],
        out_shape=[
            jax.ShapeDtypeStruct((N, H), jnp.float32),
            jax.ShapeDtypeStruct((N, H), jnp.float32),
            jax.ShapeDtypeStruct((N, 8), jnp.float32),
        ],
    )(x_e, W_h, W_t, Ah, At)


def _rel_body(nt_ref, re_ref, ar_ref, at2_ref, at3_ref, ac_ref, rt_ref, sv_ref,
              ntt_ref):
    rt = jax.lax.dot_general(ar_ref[...], re_ref[...], (((0,), (1,)), ((), ())),
                             preferred_element_type=jnp.float32)
    rt_ref[...] = rt
    nt = nt_ref[...].T
    ntt_ref[...] = nt
    mq1, mq2, mq3, mq4 = (jnp.max(nt[0]), jnp.max(nt[1]), jnp.max(nt[2]),
                          jnp.max(nt[3]))
    mt1 = jnp.max(nt[4])
    mr1 = jnp.max(rt[0])
    mr2 = jnp.max(rt[1])
    nt2 = jnp.sqrt(jnp.sum(at2_ref[...] ** 2))
    nt3 = jnp.sqrt(jnp.sum(at3_ref[...] ** 2))
    nac = jnp.sqrt(jnp.sum(ac_ref[...] ** 2))
    s1 = _lk((mq1 + mt1) / 2.0 + mr1)
    s2 = _lk(nt2 + mq2 + (mr2 + (mq3 + nt3) / 2.0) / 2.0)
    s3 = _lk(nac + mq4)
    sv = jnp.concatenate([jnp.stack([s1, s2, s3]), jnp.zeros((5,), jnp.float32)])
    sv_ref[...] = jnp.broadcast_to(sv[:, None], (8, 16))


def _rel_call(ntab, r_emb, Ar, a_t2, a_t3, a_c):
    return pl.pallas_call(
        _rel_body,
        in_specs=[
            pl.BlockSpec((N, 8), lambda: (0, 0)),
            pl.BlockSpec((R, H), lambda: (0, 0)),
            pl.BlockSpec((H, 2), lambda: (0, 0)),
            pl.BlockSpec((H,), lambda: (0,)),
            pl.BlockSpec((H,), lambda: (0,)),
            pl.BlockSpec((H,), lambda: (0,)),
        ],
        out_specs=[
            pl.BlockSpec((2, R), lambda: (0, 0)),
            pl.BlockSpec((8, 16), lambda: (0, 0)),
            pl.BlockSpec((8, N), lambda: (0, 0)),
        ],
        out_shape=[
            jax.ShapeDtypeStruct((2, R), jnp.float32),
            jax.ShapeDtypeStruct((8, 16), jnp.float32),
            jax.ShapeDtypeStruct((8, N), jnp.float32),
        ],
    )(ntab, r_emb, Ar, a_t2, a_t3, a_c)


def _edge1_body(eh_hbm, et_hbm, rel_hbm, tn_hbm, ntt_hbm, rtab_hbm, svec_hbm,
                u1_hbm, sumA_hbm,
                q1_v, t1_v, r1_v, sv_v, eh_v, et_v, rel_v, tn_v, u1_v, zz_v,
                sumA_sp):
    cid = lax.axis_index("c")
    sid = lax.axis_index("s")
    wid = sid * NC + cid
    pltpu.sync_copy(ntt_hbm.at[pl.ds(0, N)], q1_v)
    pltpu.sync_copy(ntt_hbm.at[pl.ds(4 * N, N)], t1_v)
    pltpu.sync_copy(rtab_hbm.at[pl.ds(0, R)], r1_v)
    pltpu.sync_copy(svec_hbm.at[pl.ds(0, L)], sv_v)

    def _zf(i, _):
        zz_v[pl.ds(i * L, L)] = jnp.zeros((L,), jnp.float32)
        return 0
    lax.fori_loop(0, NSEG // L, _zf, 0)
    pltpu.sync_copy(zz_v, sumA_sp.at[pl.ds(sid * NSEG, NSEG)])
    plsc.subcore_barrier()
    S1 = sv_v[...]
    base_row = wid * WR

    def _chunk(ch, _):
        r0 = base_row + ch * CHR
        pltpu.sync_copy(eh_hbm.at[pl.ds(r0, CHR)], eh_v)
        pltpu.sync_copy(et_hbm.at[pl.ds(r0, CHR)], et_v)
        pltpu.sync_copy(rel_hbm.at[pl.ds(r0, CHR)], rel_v)
        pltpu.sync_copy(tn_hbm.at[pl.ds(r0, CHR)], tn_v)

        def _row(r, _):
            for l in range(8):
                ehx = eh_v[r, pl.ds(l * L, L)]
                etx = et_v[r, pl.ds(l * L, L)]
                rlx = rel_v[r, pl.ds(l * L, L)]
                e1 = (plsc.load_gather(q1_v, [ehx])
                      + plsc.load_gather(t1_v, [etx])) * 0.5 \
                     + plsc.load_gather(r1_v, [rlx])
                z = jnp.where(e1 >= 0, e1, 0.01 * e1)
                u1_v[r, pl.ds(l * L, L)] = jnp.exp(z - S1)
            return 0
        lax.fori_loop(0, CHR, _row, 0)
        pltpu.sync_copy(u1_v, u1_hbm.at[pl.ds(r0, CHR)])

        def _srow(r, _):
            pltpu.sync_copy(u1_v.at[r], sumA_sp.at[tn_v.at[r]], add=True)
            return 0
        lax.fori_loop(0, CHR, _srow, 0)
        return 0
    lax.fori_loop(0, EW // (CHR * 128), _chunk, 0)
    plsc.subcore_barrier()
    pltpu.sync_copy(sumA_sp.at[pl.ds(sid * NSEG, NSEG)],
                    sumA_hbm.at[pl.ds(cid * N_PAD + sid * NSEG, NSEG)])


def _edge1_call(eh2, et2, rel2, tn2, ntt, rtab, svec):
    f = pl.kernel(
        _edge1_body,
        out_type=[
            jax.ShapeDtypeStruct((ER, 128), jnp.float32),
            jax.ShapeDtypeStruct((NC * N_PAD,), jnp.float32),
        ],
        mesh=plsc.VectorSubcoreMesh(core_axis_name="c", subcore_axis_name="s"),
        compiler_params=pltpu.CompilerParams(needs_layout_passes=False),
        scratch_types=[
            pltpu.VMEM((N,), jnp.float32),
            pltpu.VMEM((N,), jnp.float32),
            pltpu.VMEM((R,), jnp.float32),
            pltpu.VMEM((L,), jnp.float32),
            pltpu.VMEM((CHR, 128), jnp.int32),
            pltpu.VMEM((CHR, 128), jnp.int32),
            pltpu.VMEM((CHR, 128), jnp.int32),
            pltpu.VMEM((CHR, 128), jnp.int32),
            pltpu.VMEM((CHR, 128), jnp.float32),
            pltpu.VMEM((NSEG,), jnp.float32),
            pltpu.VMEM_SHARED((N_PAD,), jnp.float32),
        ],
    )
    return f(eh2, et2, rel2, tn2, ntt, rtab, svec)


def _edge2_body(eh_hbm, et_hbm, rel_hbm, tn_hbm, ci_hbm, u1_hbm,
                ntt_hbm, rtab_hbm, svec_hbm, sA_hbm,
                v_hbm, sumB_hbm,
                q2_v, q3_v, t2_v, t3_v, nrm_v, r2_v, sv_v, sa0_v, sa1_v,
                eh_v, et_v, rel_v, tn_v, ci_v, u1_v, u2_v, v_v, zz_v,
                sumB_sp):
    cid = lax.axis_index("c")
    sid = lax.axis_index("s")
    wid = sid * NC + cid
    pltpu.sync_copy(ntt_hbm.at[pl.ds(1 * N, N)], q2_v)
    pltpu.sync_copy(ntt_hbm.at[pl.ds(2 * N, N)], q3_v)
    pltpu.sync_copy(ntt_hbm.at[pl.ds(5 * N, N)], t2_v)
    pltpu.sync_copy(ntt_hbm.at[pl.ds(6 * N, N)], t3_v)
    pltpu.sync_copy(ntt_hbm.at[pl.ds(7 * N, N)], nrm_v)
    pltpu.sync_copy(rtab_hbm.at[pl.ds(R, R)], r2_v)
    pltpu.sync_copy(svec_hbm.at[pl.ds(L, L)], sv_v)
    pltpu.sync_copy(sA_hbm.at[pl.ds(0, N_PAD)], sa0_v)
    pltpu.sync_copy(sA_hbm.at[pl.ds(N_PAD, N_PAD)], sa1_v)

    def _zf(i, _):
        zz_v[pl.ds(i * L, L)] = jnp.zeros((L,), jnp.float32)
        return 0
    lax.fori_loop(0, NSEG // L, _zf, 0)

    @pl.when(sid < 8)
    def _zero_sumb():
        pltpu.sync_copy(zz_v, sumB_sp.at[pl.ds(sid * NSEG, NSEG)])
    plsc.subcore_barrier()

    S2 = sv_v[...]
    base_row = wid * WR

    def _chunk(ch, _):
        r0 = base_row + ch * CHR
        pltpu.sync_copy(eh_hbm.at[pl.ds(r0, CHR)], eh_v)
        pltpu.sync_copy(et_hbm.at[pl.ds(r0, CHR)], et_v)
        pltpu.sync_copy(rel_hbm.at[pl.ds(r0, CHR)], rel_v)
        pltpu.sync_copy(tn_hbm.at[pl.ds(r0, CHR)], tn_v)
        pltpu.sync_copy(ci_hbm.at[pl.ds(r0, CHR)], ci_v)
        pltpu.sync_copy(u1_hbm.at[pl.ds(r0, CHR)], u1_v)

        def _row(r, _):
            for l in range(8):
                ehx = eh_v[r, pl.ds(l * L, L)]
                etx = et_v[r, pl.ds(l * L, L)]
                rlx = rel_v[r, pl.ds(l * L, L)]
                tnx = tn_v[r, pl.ds(l * L, L)]
                u1x = u1_v[r, pl.ds(l * L, L)]
                sa = (plsc.load_gather(sa0_v, [tnx])
                      + plsc.load_gather(sa1_v, [tnx]))
                alpha = u1x / (sa + 1e-38)
                nr = plsc.load_gather(nrm_v, [etx])
                s = alpha / jnp.maximum(nr * alpha, 1e-12)
                e = (s * plsc.load_gather(t2_v, [etx])
                     + plsc.load_gather(q2_v, [ehx])
                     + (plsc.load_gather(r2_v, [rlx])
                        + (plsc.load_gather(q3_v, [ehx])
                           + s * plsc.load_gather(t3_v, [etx])) * 0.5) * 0.5)
                z = jnp.where(e >= 0, e, 0.01 * e)
                u2 = jnp.exp(z - S2)
                u2_v[r, pl.ds(l * L, L)] = u2
                v_v[r, pl.ds(l * L, L)] = s * u2
            return 0
        lax.fori_loop(0, CHR, _row, 0)
        pltpu.sync_copy(v_v, v_hbm.at[pl.ds(r0, CHR)])

        def _srow(r, _):
            pltpu.sync_copy(u2_v.at[r], sumB_sp.at[ci_v.at[r]], add=True)
            return 0
        lax.fori_loop(0, CHR, _srow, 0)
        return 0
    lax.fori_loop(0, EW // (CHR * 128), _chunk, 0)
    plsc.subcore_barrier()

    @pl.when(sid < 8)
    def _write_sumb():
        pltpu.sync_copy(sumB_sp.at[pl.ds(sid * NSEG, NSEG)],
                        sumB_hbm.at[pl.ds(cid * C_PAD + sid * NSEG, NSEG)])


def _edge2_call(eh2, et2, rel2, tn2, ci2, u1_2, ntt, rtab, svec, sumA_p):
    f = pl.kernel(
        _edge2_body,
        out_type=[
            jax.ShapeDtypeStruct((ER, 128), jnp.float32),
            jax.ShapeDtypeStruct((NC * C_PAD,), jnp.float32),
        ],
        mesh=plsc.VectorSubcoreMesh(core_axis_name="c", subcore_axis_name="s"),
        compiler_params=pltpu.CompilerParams(needs_layout_passes=False),
        scratch_types=[
            pltpu.VMEM((N,), jnp.float32),
            pltpu.VMEM((N,), jnp.float32),
            pltpu.VMEM((N,), jnp.float32),
            pltpu.VMEM((N,), jnp.float32),
            pltpu.VMEM((N,), jnp.float32),
            pltpu.VMEM((R,), jnp.float32),
            pltpu.VMEM((L,), jnp.float32),
            pltpu.VMEM((N_PAD,), jnp.float32),
            pltpu.VMEM((N_PAD,), jnp.float32),
            pltpu.VMEM((CHR, 128), jnp.int32),
            pltpu.VMEM((CHR, 128), jnp.int32),
            pltpu.VMEM((CHR, 128), jnp.int32),
            pltpu.VMEM((CHR, 128), jnp.int32),
            pltpu.VMEM((CHR, 128), jnp.int32),
            pltpu.VMEM((CHR, 128), jnp.float32),
            pltpu.VMEM((CHR, 128), jnp.float32),
            pltpu.VMEM((CHR, 128), jnp.float32),
            pltpu.VMEM((NSEG,), jnp.float32),
            pltpu.VMEM_SHARED((C_PAD,), jnp.float32),
        ],
    )
    return f(eh2, et2, rel2, tn2, ci2, u1_2, ntt, rtab, svec, sumA_p)


def _edge3_body(et_hbm, ci_hbm, v_hbm, xrt_hbm, sumB_hbm,
                acc_hbm,
                sb0_v, sb1_v, et_v, ci_v, v_v, w_v, rows_v, rows1_v, rows2_v,
                rows3_v, zz_v, acc_sp, sem, ssem):
    cid = lax.axis_index("c")
    sid = lax.axis_index("s")
    wid = sid * NC + cid
    pltpu.sync_copy(sumB_hbm.at[pl.ds(0, C_PAD)], sb0_v)
    pltpu.sync_copy(sumB_hbm.at[pl.ds(C_PAD, C_PAD)], sb1_v)

    def _zr(r, _):
        for l in range(8):
            zz_v[r, pl.ds(l * L, L)] = jnp.zeros((L,), jnp.float32)
        return 0
    lax.fori_loop(0, 16, _zr, 0)

    def _zc(i, _):
        pltpu.sync_copy(zz_v, acc_sp.at[pl.ds(sid * CSEG + i * 16, 16)])
        return 0
    lax.fori_loop(0, CSEG // 16, _zc, 0)
    plsc.subcore_barrier()

    nch = jnp.where(cid == FAST_CID, FCH, SCH)
    base_row = jnp.where(cid == FAST_CID, sid * FCH * CHR,
                         NS * FCH * CHR + sid * SCH * CHR)

    def _chunk(ch, _):
        r0 = base_row + ch * CHR
        pltpu.sync_copy(et_hbm.at[pl.ds(r0, CHR)], et_v)
        pltpu.sync_copy(ci_hbm.at[pl.ds(r0, CHR)], ci_v)
        pltpu.sync_copy(v_hbm.at[pl.ds(r0, CHR)], v_v)

        def _wrow(r, _):
            for l in range(8):
                cix = ci_v[r, pl.ds(l * L, L)]
                sb = (plsc.load_gather(sb0_v, [cix])
                      + plsc.load_gather(sb1_v, [cix]))
                w_v[r, pl.ds(l * L, L)] = v_v[r, pl.ds(l * L, L)] / (sb + 1e-38)
            return 0
        lax.fori_loop(0, CHR, _wrow, 0)

        def _scale_row(rows_b, r):
            def _sg(g, _):
                for i in range(L):
                    e = g * L + i
                    wb = plsc.load_gather(
                        w_v, [jnp.full((L,), r, jnp.int32),
                              jnp.full((L,), e, jnp.int32)])
                    for l in range(8):
                        rows_b[e, pl.ds(l * L, L)] = (
                            rows_b[e, pl.ds(l * L, L)] * wb)
                return 0
            lax.fori_loop(0, 128 // L, _sg, 0)

        bufs = (rows_v, rows1_v, rows2_v, rows3_v)

        def _drain(sm):
            pltpu.make_async_copy(xrt_hbm.at[pl.ds(0, 128)], rows_v, sm).wait()

        pltpu.async_copy(xrt_hbm.at[et_v.at[0]], bufs[0], sem)
        pltpu.async_copy(xrt_hbm.at[et_v.at[1]], bufs[1], sem)

        def _quad(q, _):
            for i in range(4):
                r = 4 * q + i

                @pl.when(r >= 2)
                def _drain_scat():
                    _drain(ssem)

                @pl.when(r + 2 < CHR)
                def _fire_gather():
                    pltpu.async_copy(xrt_hbm.at[et_v.at[r + 2]],
                                     bufs[(i + 2) % 4], sem)
                _drain(sem)
                _scale_row(bufs[i], r)
                pltpu.async_copy(bufs[i], acc_sp.at[ci_v.at[r]], ssem,
                                 add=True)
            return 0
        lax.fori_loop(0, CHR // 4, _quad, 0)
        _drain(ssem)
        _drain(ssem)
        return 0
    lax.fori_loop(0, nch, _chunk, 0)
    plsc.subcore_barrier()
    pltpu.sync_copy(acc_sp.at[pl.ds(sid * CSEG, CSEG)],
                    acc_hbm.at[pl.ds(cid * C_PAD + sid * CSEG, CSEG)])


def _edge3_call(et2, ci2, v2, x_r_t, sumB_p):
    f = pl.kernel(
        _edge3_body,
        out_type=[
            jax.ShapeDtypeStruct((NC * C_PAD, H), jnp.float32),
        ],
        mesh=plsc.VectorSubcoreMesh(core_axis_name="c", subcore_axis_name="s"),
        compiler_params=pltpu.CompilerParams(needs_layout_passes=False),
        scratch_types=[
            pltpu.VMEM((C_PAD,), jnp.float32),
            pltpu.VMEM((C_PAD,), jnp.float32),
            pltpu.VMEM((CHR, 128), jnp.int32),
            pltpu.VMEM((CHR, 128), jnp.int32),
            pltpu.VMEM((CHR, 128), jnp.float32),
            pltpu.VMEM((CHR, 128), jnp.float32),
            pltpu.VMEM((128, H), jnp.float32),
            pltpu.VMEM((128, H), jnp.float32),
            pltpu.VMEM((128, H), jnp.float32),
            pltpu.VMEM((128, H), jnp.float32),
            pltpu.VMEM((16, H), jnp.float32),
            pltpu.VMEM_SHARED((C_PAD, H), jnp.float32),
            pltpu.SemaphoreType.DMA,
            pltpu.SemaphoreType.DMA,
        ],
    )
    return f(et2, ci2, v2, x_r_t, sumB_p)[0]


CW = C_PAD // NW
CWG = CW // L


def _cls1_body(acc_hbm, hc_hbm, ntt_hbm, ac_hbm, svec_hbm,
               xcls_hbm, u3_hbm, sumC_hbm,
               q4_v, ac_v, sv_v, hc_v, hc1_v, u3_v, u3f_v, ec_v, xc_v, xc1_v,
               zz_v, sumC_sp):
    cid = lax.axis_index("c")
    sid = lax.axis_index("s")
    wid = sid * NC + cid
    pltpu.sync_copy(ntt_hbm.at[pl.ds(3 * N, N)], q4_v)
    pltpu.sync_copy(ac_hbm, ac_v)
    pltpu.sync_copy(svec_hbm.at[pl.ds(2 * L, L)], sv_v)
    pltpu.sync_copy(hc_hbm.at[pl.ds(wid * 256, 256)], hc1_v)

    def _hcb(g, _):
        hc_v[g] = hc1_v[pl.ds(g * L, L)]
        return 0
    lax.fori_loop(0, CWG, _hcb, 0)

    def _zf(i, _):
        zz_v[pl.ds(i * L, L)] = jnp.zeros((L,), jnp.float32)
        return 0
    lax.fori_loop(0, NSEG // L, _zf, 0)
    pltpu.sync_copy(zz_v, sumC_sp.at[pl.ds(sid * NSEG, NSEG)])
    plsc.subcore_barrier()

    pltpu.sync_copy(acc_hbm.at[pl.ds(wid * CW, CW)], xc_v)
    pltpu.sync_copy(acc_hbm.at[pl.ds(C_PAD + wid * CW, CW)], xc1_v)

    def _addr(r, _):
        for l in range(8):
            xc_v[r, pl.ds(l * L, L)] = (xc_v[r, pl.ds(l * L, L)]
                                        + xc1_v[r, pl.ds(l * L, L)])
        return 0
    lax.fori_loop(0, CW, _addr, 0)
    pltpu.sync_copy(xc_v, xcls_hbm.at[pl.ds(wid * CW, CW)])

    def _dotg(g, _):
        rows16 = g * L + lax.broadcasted_iota(jnp.int32, (L,), 0)

        def _j(j, acc):
            colv = plsc.load_gather(xc_v, [rows16, jnp.full((L,), j, jnp.int32)])
            acb = plsc.load_gather(ac_v, [jnp.full((L,), j, jnp.int32)])
            return acc + colv * acb
        ec_v[g] = lax.fori_loop(0, H, _j, jnp.zeros((L,), jnp.float32))
        return 0
    lax.fori_loop(0, CWG, _dotg, 0)

    S3 = sv_v[...]

    def _grp(g, _):
        hcx = hc_v[g]
        q4g = plsc.load_gather(q4_v, [jnp.minimum(hcx, N - 1)])
        e_c = ec_v[g] + q4g
        z = jnp.where(e_c >= 0, e_c, 0.01 * e_c)
        u3g = jnp.where(hcx < N, jnp.exp(z - S3), 0.0)
        u3_v[g] = u3g
        u3f_v[pl.ds(g * L, L)] = u3g
        pltpu.sync_copy(u3_v.at[g], sumC_sp.at[hc_v.at[g]], add=True)
        return 0
    lax.fori_loop(0, CWG, _grp, 0)
    pltpu.sync_copy(u3f_v, u3_hbm.at[pl.ds(wid * 256, 256)])
    plsc.subcore_barrier()
    pltpu.sync_copy(sumC_sp.at[pl.ds(sid * NSEG, NSEG)],
                    sumC_hbm.at[pl.ds(cid * N_PAD + sid * NSEG, NSEG)])


def _cls1_call(acc_p, hc2, ntt, a_c, svec):
    f = pl.kernel(
        _cls1_body,
        out_type=[
            jax.ShapeDtypeStruct((C_PAD, H), jnp.float32),
            jax.ShapeDtypeStruct((NW * 256,), jnp.float32),
            jax.ShapeDtypeStruct((NC * N_PAD,), jnp.float32),
        ],
        mesh=plsc.VectorSubcoreMesh(core_axis_name="c", subcore_axis_name="s"),
        compiler_params=pltpu.CompilerParams(needs_layout_passes=False),
        scratch_types=[
            pltpu.VMEM((N,), jnp.float32),
            pltpu.VMEM((H,), jnp.float32),
            pltpu.VMEM((L,), jnp.float32),
            pltpu.VMEM((L, L), jnp.int32),
            pltpu.VMEM((256,), jnp.int32),
            pltpu.VMEM((L, L), jnp.float32),
            pltpu.VMEM((256,), jnp.float32),
            pltpu.VMEM((L, L), jnp.float32),
            pltpu.VMEM((CW, H), jnp.float32),
            pltpu.VMEM((CW, H), jnp.float32),
            pltpu.VMEM((NSEG,), jnp.float32),
            pltpu.VMEM_SHARED((N_PAD,), jnp.float32),
        ],
    )
    return f(acc_p, hc2, ntt, a_c, svec)


def _cls2_body(xcls_hbm, u3_hbm, hc_hbm, sumC_hbm,
               xeh_hbm,
               sc0_v, sc1_v, hc_v, hc1_v, u3f_v, gm_v, xc_v, zz_v,
               xeh_sp):
    cid = lax.axis_index("c")
    sid = lax.axis_index("s")
    wid = sid * NC + cid
    pltpu.sync_copy(sumC_hbm.at[pl.ds(0, N_PAD)], sc0_v)
    pltpu.sync_copy(sumC_hbm.at[pl.ds(N_PAD, N_PAD)], sc1_v)
    pltpu.sync_copy(hc_hbm.at[pl.ds(wid * 256, 256)], hc1_v)
    pltpu.sync_copy(u3_hbm.at[pl.ds(wid * 256, 256)], u3f_v)
    pltpu.sync_copy(xcls_hbm.at[pl.ds(wid * CW, CW)], xc_v)

    def _hcb(g, _):
        hc_v[g] = hc1_v[pl.ds(g * L, L)]
        return 0
    lax.fori_loop(0, CWG, _hcb, 0)

    def _zr(r, _):
        for l in range(8):
            zz_v[r, pl.ds(l * L, L)] = jnp.zeros((L,), jnp.float32)
        return 0
    lax.fori_loop(0, 16, _zr, 0)

    def _zc(i, _):
        pltpu.sync_copy(zz_v, xeh_sp.at[pl.ds(sid * NSEG + i * 16, 16)])
        return 0
    lax.fori_loop(0, NSEG // 16, _zc, 0)
    plsc.subcore_barrier()

    def _grp(g, _):
        hcx = hc_v[g]
        sc = plsc.load_gather(sc0_v, [hcx]) + plsc.load_gather(sc1_v, [hcx])
        gm_v[g] = u3f_v[pl.ds(g * L, L)] / (sc + 1e-38)
        return 0
    lax.fori_loop(0, CWG, _grp, 0)

    def _scale(r, _):
        wb = plsc.load_gather(
            gm_v, [jnp.full((L,), r // L, jnp.int32),
                   jnp.full((L,), r % L, jnp.int32)])
        for l in range(8):
            xc_v[r, pl.ds(l * L, L)] = xc_v[r, pl.ds(l * L, L)] * wb
        return 0
    lax.fori_loop(0, CW, _scale, 0)

    def _scat(g, _):
        pltpu.sync_copy(xc_v.at[pl.ds(g * L, L)], xeh_sp.at[hc_v.at[g]],
                        add=True)
        return 0
    lax.fori_loop(0, CWG, _scat, 0)
    plsc.subcore_barrier()
    pltpu.sync_copy(xeh_sp.at[pl.ds(sid * NSEG, NSEG)],
                    xeh_hbm.at[pl.ds(cid * N_PAD + sid * NSEG, NSEG)])


def _cls2_call(xcls, u3, hc2, sumC_p):
    f = pl.kernel(
        _cls2_body,
        out_type=[
            jax.ShapeDtypeStruct((NC * N_PAD, H), jnp.float32),
        ],
        mesh=plsc.VectorSubcoreMesh(core_axis_name="c", subcore_axis_name="s"),
        compiler_params=pltpu.CompilerParams(needs_layout_passes=False),
        scratch_types=[
            pltpu.VMEM((N_PAD,), jnp.float32),
            pltpu.VMEM((N_PAD,), jnp.float32),
            pltpu.VMEM((L, L), jnp.int32),
            pltpu.VMEM((256,), jnp.int32),
            pltpu.VMEM((256,), jnp.float32),
            pltpu.VMEM((L, L), jnp.float32),
            pltpu.VMEM((CW, H), jnp.float32),
            pltpu.VMEM((16, H), jnp.float32),
            pltpu.VMEM_SHARED((N_PAD, H), jnp.float32),
        ],
    )
    return f(xcls, u3, hc2, sumC_p)[0]


def _gate_body(h_ref, e0_ref, e1_ref, w_ref, b_ref, o_ref):
    h = h_ref[...]
    g = jax.nn.sigmoid(jax.lax.dot(h, w_ref[...], preferred_element_type=jnp.float32)
                       + b_ref[...][None, :])
    o_ref[...] = g * (e0_ref[...] + e1_ref[...]) + (1.0 - g) * h


def _gate_call(x_r_h, xeh0, xeh1, hw_W, hw_b):
    return pl.pallas_call(
        _gate_body,
        grid=(N // NB,),
        in_specs=[
            pl.BlockSpec((NB, H), lambda i: (i, 0)),
            pl.BlockSpec((NB, H), lambda i: (i, 0)),
            pl.BlockSpec((NB, H), lambda i: (i, 0)),
            pl.BlockSpec((H, H), lambda i: (0, 0)),
            pl.BlockSpec((H,), lambda i: (0,)),
        ],
        out_specs=pl.BlockSpec((NB, H), lambda i: (i, 0)),
        out_shape=jax.ShapeDtypeStruct((N, H), jnp.float32),
    )(x_r_h, xeh0, xeh1, hw_W, hw_b)


def kernel(x_e, edge_index, rel, triple_num, r_emb, class_index, head_class,
           a_h1, a_h2, a_h3, a_h4, a_t1, a_t2, a_t3, a_r1, a_r2, a_c,
           W_h, W_t, hw_W, hw_b):
    Ah = jnp.stack([a_h1, a_h2, a_h3, a_h4], axis=1)
    At = jnp.stack([a_t1, a_t2, a_t3], axis=1)
    Ar = jnp.stack([a_r1, a_r2], axis=1)
    x_r_h, x_r_t, ntab = _nodes_call(x_e, W_h, W_t, Ah, At)
    rtab, svec, ntt = _rel_call(ntab, r_emb, Ar, a_t2, a_t3, a_c)
    S1, S2, S3 = svec[0, 0], svec[1, 0], svec[2, 0]
    q1, q2, q3, q4 = ntt[0], ntt[1], ntt[2], ntt[3]
    t1, t2, t3, nrm = ntt[4], ntt[5], ntt[6], ntt[7]
    r1, r2 = rtab[0], rtab[1]
    eh, et = edge_index[0], edge_index[1]
    tn, ci, hc = triple_num, class_index, head_class
    EPS = 1e-38

    pad_i = jnp.zeros((E_PAD - E,), jnp.int32)
    eh2 = jnp.concatenate([eh, pad_i]).reshape(ER, 128)
    et2 = jnp.concatenate([et, pad_i]).reshape(ER, 128)
    rel2 = jnp.concatenate([rel, pad_i]).reshape(ER, 128)
    tn2 = jnp.concatenate([tn, jnp.full((E_PAD - E,), N_PAD - 1, jnp.int32)]
                          ).reshape(ER, 128)
    ci2 = jnp.concatenate([ci, jnp.full((E_PAD - E,), C_PAD - 1, jnp.int32)]
                          ).reshape(ER, 128)
    hcp = jnp.concatenate([hc, jnp.full((C_PAD - C,), N_PAD - 1, jnp.int32)])
    hc3 = jnp.pad(hcp.reshape(NW, CW), ((0, 0), (0, 256 - CW)),
                  constant_values=N_PAD - 1).reshape(NW * 256)
    ntt_f = ntt.reshape(8 * N)
    rtab_f = rtab.reshape(2 * R)
    svec_f = svec.reshape(8 * L)

    u1_2, sumA_p = _edge1_call(eh2, et2, rel2, tn2, ntt_f, rtab_f, svec_f)
    v2, sumB_p = _edge2_call(eh2, et2, rel2, tn2, ci2, u1_2, ntt_f, rtab_f,
                             svec_f, sumA_p)
    acc_p = _edge3_call(et2, ci2, v2, x_r_t, sumB_p)
    xcls, u3, sumC_p = _cls1_call(acc_p, hc3, ntt_f, a_c, svec_f)
    xeh_p = _cls2_call(xcls, u3, hc3, sumC_p)

    return _gate_call(x_r_h, xeh_p[:N], xeh_p[N_PAD:N_PAD + N], hw_W, hw_b)


# speedup vs baseline: 1.2207x; 1.0124x over previous
import functools
import jax
import jax.numpy as jnp
from jax import lax
from jax.experimental import pallas as pl
from jax.experimental.pallas import tpu as pltpu
from jax.experimental.pallas import tpu_sc as plsc

N = 10000
E = 320000
H = 128
R = 1000
C = 5000

NB = 2000
N_PAD = 10240
C_PAD = 5120
E_PAD = 327680

NC = 2
NS = 16
L = 16
NW = NC * NS
EW = E_PAD // NW
ER = E_PAD // 128
WR = EW // 128
CHR = 16
FAST_CID = 1
FCH = 7
SCH = 3
NSEG = N_PAD // NS
CSEG = C_PAD // NS

_ZIDX = functools.partial(jnp.full, (L,), dtype=jnp.int32)


def _bcast0(ref):
    return plsc.load_gather(ref, [jnp.zeros((L,), jnp.int32)])


def _lk(x):
    return jnp.where(x >= 0, x, 0.01 * x)


def _nodes_body(x_ref, wh_ref, wt_ref, ah_ref, at_ref, xh_ref, xt_ref, nt_ref):
    x = x_ref[...]
    h = jax.nn.relu(jax.lax.dot(x, wh_ref[...], preferred_element_type=jnp.float32))
    t = jax.nn.relu(jax.lax.dot(x, wt_ref[...], preferred_element_type=jnp.float32))
    xh_ref[...] = h
    xt_ref[...] = t
    sh = jax.lax.dot_general(ah_ref[...], h, (((0,), (1,)), ((), ())),
                             preferred_element_type=jnp.float32)
    st = jax.lax.dot_general(at_ref[...], t, (((0,), (1,)), ((), ())),
                             preferred_element_type=jnp.float32)
    nrm = jnp.sqrt(jnp.sum(t * t, axis=1))[None, :]
    nt_ref[...] = jnp.concatenate([sh, st, nrm], axis=0).T


def _nodes_call(x_e, W_h, W_t, Ah, At):
    return pl.pallas_call(
        _nodes_body,
        grid=(N // NB,),
        in_specs=[
            pl.BlockSpec((NB, H), lambda i: (i, 0)),
            pl.BlockSpec((H, H), lambda i: (0, 0)),
            pl.BlockSpec((H, H), lambda i: (0, 0)),
            pl.BlockSpec((H, 4), lambda i: (0, 0)),
            pl.BlockSpec((H, 3), lambda i: (0, 0)),
        ],
        out_specs=[
            pl.BlockSpec((NB, H), lambda i: (i, 0)),
            pl.BlockSpec((NB, H), lambda i: (i, 0)),
            pl.BlockSpec((NB, 8), lambda i: (i, 0)),
        ],
        out_shape=[
            jax.ShapeDtypeStruct((N, H), jnp.float32),
            jax.ShapeDtypeStruct((N, H), jnp.float32),
            jax.ShapeDtypeStruct((N, 8), jnp.float32),
        ],
    )(x_e, W_h, W_t, Ah, At)


def _rel_body(nt_ref, re_ref, ar_ref, at2_ref, at3_ref, ac_ref, rt_ref, sv_ref,
              ntt_ref):
    rt = jax.lax.dot_general(ar_ref[...], re_ref[...], (((0,), (1,)), ((), ())),
                             preferred_element_type=jnp.float32)
    rt_ref[...] = rt
    nt = nt_ref[...].T
    ntt_ref[...] = nt
    mq1, mq2, mq3, mq4 = (jnp.max(nt[0]), jnp.max(nt[1]), jnp.max(nt[2]),
                          jnp.max(nt[3]))
    mt1 = jnp.max(nt[4])
    mr1 = jnp.max(rt[0])
    mr2 = jnp.max(rt[1])
    nt2 = jnp.sqrt(jnp.sum(at2_ref[...] ** 2))
    nt3 = jnp.sqrt(jnp.sum(at3_ref[...] ** 2))
    nac = jnp.sqrt(jnp.sum(ac_ref[...] ** 2))
    s1 = _lk((mq1 + mt1) / 2.0 + mr1)
    s2 = _lk(nt2 + mq2 + (mr2 + (mq3 + nt3) / 2.0) / 2.0)
    s3 = _lk(nac + mq4)
    sv = jnp.concatenate([jnp.stack([s1, s2, s3]), jnp.zeros((5,), jnp.float32)])
    sv_ref[...] = jnp.broadcast_to(sv[:, None], (8, 16))


def _rel_call(ntab, r_emb, Ar, a_t2, a_t3, a_c):
    return pl.pallas_call(
        _rel_body,
        in_specs=[
            pl.BlockSpec((N, 8), lambda: (0, 0)),
            pl.BlockSpec((R, H), lambda: (0, 0)),
            pl.BlockSpec((H, 2), lambda: (0, 0)),
            pl.BlockSpec((H,), lambda: (0,)),
            pl.BlockSpec((H,), lambda: (0,)),
            pl.BlockSpec((H,), lambda: (0,)),
        ],
        out_specs=[
            pl.BlockSpec((2, R), lambda: (0, 0)),
            pl.BlockSpec((8, 16), lambda: (0, 0)),
            pl.BlockSpec((8, N), lambda: (0, 0)),
        ],
        out_shape=[
            jax.ShapeDtypeStruct((2, R), jnp.float32),
            jax.ShapeDtypeStruct((8, 16), jnp.float32),
            jax.ShapeDtypeStruct((8, N), jnp.float32),
        ],
    )(ntab, r_emb, Ar, a_t2, a_t3, a_c)


def _edge1_body(eh_hbm, et_hbm, rel_hbm, tn_hbm, ntt_hbm, rtab_hbm, svec_hbm,
                u1_hbm, sumA_hbm,
                q1_v, t1_v, r1_v, sv_v, eh_v, et_v, rel_v, tn_v, u1_v, zz_v,
                sumA_sp):
    cid = lax.axis_index("c")
    sid = lax.axis_index("s")
    wid = sid * NC + cid
    pltpu.sync_copy(ntt_hbm.at[pl.ds(0, N)], q1_v)
    pltpu.sync_copy(ntt_hbm.at[pl.ds(4 * N, N)], t1_v)
    pltpu.sync_copy(rtab_hbm.at[pl.ds(0, R)], r1_v)
    pltpu.sync_copy(svec_hbm.at[pl.ds(0, L)], sv_v)

    def _zf(i, _):
        zz_v[pl.ds(i * L, L)] = jnp.zeros((L,), jnp.float32)
        return 0
    lax.fori_loop(0, NSEG // L, _zf, 0)
    pltpu.sync_copy(zz_v, sumA_sp.at[pl.ds(sid * NSEG, NSEG)])
    plsc.subcore_barrier()
    S1 = sv_v[...]
    base_row = wid * WR

    def _chunk(ch, _):
        r0 = base_row + ch * CHR
        pltpu.sync_copy(eh_hbm.at[pl.ds(r0, CHR)], eh_v)
        pltpu.sync_copy(et_hbm.at[pl.ds(r0, CHR)], et_v)
        pltpu.sync_copy(rel_hbm.at[pl.ds(r0, CHR)], rel_v)
        pltpu.sync_copy(tn_hbm.at[pl.ds(r0, CHR)], tn_v)

        def _row(r, _):
            for l in range(8):
                ehx = eh_v[r, pl.ds(l * L, L)]
                etx = et_v[r, pl.ds(l * L, L)]
                rlx = rel_v[r, pl.ds(l * L, L)]
                e1 = (plsc.load_gather(q1_v, [ehx])
                      + plsc.load_gather(t1_v, [etx])) * 0.5 \
                     + plsc.load_gather(r1_v, [rlx])
                z = jnp.where(e1 >= 0, e1, 0.01 * e1)
                u1_v[r, pl.ds(l * L, L)] = jnp.exp(z - S1)
            return 0
        lax.fori_loop(0, CHR, _row, 0)
        pltpu.sync_copy(u1_v, u1_hbm.at[pl.ds(r0, CHR)])

        def _srow(r, _):
            pltpu.sync_copy(u1_v.at[r], sumA_sp.at[tn_v.at[r]], add=True)
            return 0
        lax.fori_loop(0, CHR, _srow, 0)
        return 0
    lax.fori_loop(0, EW // (CHR * 128), _chunk, 0)
    plsc.subcore_barrier()
    pltpu.sync_copy(sumA_sp.at[pl.ds(sid * NSEG, NSEG)],
                    sumA_hbm.at[pl.ds(cid * N_PAD + sid * NSEG, NSEG)])


def _edge1_call(eh2, et2, rel2, tn2, ntt, rtab, svec):
    f = pl.kernel(
        _edge1_body,
        out_type=[
            jax.ShapeDtypeStruct((ER, 128), jnp.float32),
            jax.ShapeDtypeStruct((NC * N_PAD,), jnp.float32),
        ],
        mesh=plsc.VectorSubcoreMesh(core_axis_name="c", subcore_axis_name="s"),
        compiler_params=pltpu.CompilerParams(needs_layout_passes=False),
        scratch_types=[
            pltpu.VMEM((N,), jnp.float32),
            pltpu.VMEM((N,), jnp.float32),
            pltpu.VMEM((R,), jnp.float32),
            pltpu.VMEM((L,), jnp.float32),
            pltpu.VMEM((CHR, 128), jnp.int32),
            pltpu.VMEM((CHR, 128), jnp.int32),
            pltpu.VMEM((CHR, 128), jnp.int32),
            pltpu.VMEM((CHR, 128), jnp.int32),
            pltpu.VMEM((CHR, 128), jnp.float32),
            pltpu.VMEM((NSEG,), jnp.float32),
            pltpu.VMEM_SHARED((N_PAD,), jnp.float32),
        ],
    )
    return f(eh2, et2, rel2, tn2, ntt, rtab, svec)


def _edge2_body(eh_hbm, et_hbm, rel_hbm, tn_hbm, ci_hbm, u1_hbm,
                ntt_hbm, rtab_hbm, svec_hbm, sA_hbm,
                v_hbm, sumB_hbm,
                q2_v, q3_v, t2_v, t3_v, nrm_v, r2_v, sv_v, sa0_v, sa1_v,
                eh_v, et_v, rel_v, tn_v, ci_v, u1_v, u2_v, v_v, zz_v,
                sumB_sp):
    cid = lax.axis_index("c")
    sid = lax.axis_index("s")
    wid = sid * NC + cid
    pltpu.sync_copy(ntt_hbm.at[pl.ds(1 * N, N)], q2_v)
    pltpu.sync_copy(ntt_hbm.at[pl.ds(2 * N, N)], q3_v)
    pltpu.sync_copy(ntt_hbm.at[pl.ds(5 * N, N)], t2_v)
    pltpu.sync_copy(ntt_hbm.at[pl.ds(6 * N, N)], t3_v)
    pltpu.sync_copy(ntt_hbm.at[pl.ds(7 * N, N)], nrm_v)
    pltpu.sync_copy(rtab_hbm.at[pl.ds(R, R)], r2_v)
    pltpu.sync_copy(svec_hbm.at[pl.ds(L, L)], sv_v)
    pltpu.sync_copy(sA_hbm.at[pl.ds(0, N_PAD)], sa0_v)
    pltpu.sync_copy(sA_hbm.at[pl.ds(N_PAD, N_PAD)], sa1_v)

    def _zf(i, _):
        zz_v[pl.ds(i * L, L)] = jnp.zeros((L,), jnp.float32)
        return 0
    lax.fori_loop(0, NSEG // L, _zf, 0)

    @pl.when(sid < 8)
    def _zero_sumb():
        pltpu.sync_copy(zz_v, sumB_sp.at[pl.ds(sid * NSEG, NSEG)])
    plsc.subcore_barrier()

    S2 = sv_v[...]
    base_row = wid * WR

    def _chunk(ch, _):
        r0 = base_row + ch * CHR
        pltpu.sync_copy(eh_hbm.at[pl.ds(r0, CHR)], eh_v)
        pltpu.sync_copy(et_hbm.at[pl.ds(r0, CHR)], et_v)
        pltpu.sync_copy(rel_hbm.at[pl.ds(r0, CHR)], rel_v)
        pltpu.sync_copy(tn_hbm.at[pl.ds(r0, CHR)], tn_v)
        pltpu.sync_copy(ci_hbm.at[pl.ds(r0, CHR)], ci_v)
        pltpu.sync_copy(u1_hbm.at[pl.ds(r0, CHR)], u1_v)

        def _row(r, _):
            for l in range(8):
                ehx = eh_v[r, pl.ds(l * L, L)]
                etx = et_v[r, pl.ds(l * L, L)]
                rlx = rel_v[r, pl.ds(l * L, L)]
                tnx = tn_v[r, pl.ds(l * L, L)]
                u1x = u1_v[r, pl.ds(l * L, L)]
                sa = (plsc.load_gather(sa0_v, [tnx])
                      + plsc.load_gather(sa1_v, [tnx]))
                alpha = u1x / (sa + 1e-38)
                nr = plsc.load_gather(nrm_v, [etx])
                s = alpha / jnp.maximum(nr * alpha, 1e-12)
                e = (s * plsc.load_gather(t2_v, [etx])
                     + plsc.load_gather(q2_v, [ehx])
                     + (plsc.load_gather(r2_v, [rlx])
                        + (plsc.load_gather(q3_v, [ehx])
                           + s * plsc.load_gather(t3_v, [etx])) * 0.5) * 0.5)
                z = jnp.where(e >= 0, e, 0.01 * e)
                u2 = jnp.exp(z - S2)
                u2_v[r, pl.ds(l * L, L)] = u2
                v_v[r, pl.ds(l * L, L)] = s * u2
            return 0
        lax.fori_loop(0, CHR, _row, 0)
        pltpu.sync_copy(v_v, v_hbm.at[pl.ds(r0, CHR)])

        def _srow(r, _):
            pltpu.sync_copy(u2_v.at[r], sumB_sp.at[ci_v.at[r]], add=True)
            return 0
        lax.fori_loop(0, CHR, _srow, 0)
        return 0
    lax.fori_loop(0, EW // (CHR * 128), _chunk, 0)
    plsc.subcore_barrier()

    @pl.when(sid < 8)
    def _write_sumb():
        pltpu.sync_copy(sumB_sp.at[pl.ds(sid * NSEG, NSEG)],
                        sumB_hbm.at[pl.ds(cid * C_PAD + sid * NSEG, NSEG)])


def _edge2_call(eh2, et2, rel2, tn2, ci2, u1_2, ntt, rtab, svec, sumA_p):
    f = pl.kernel(
        _edge2_body,
        out_type=[
            jax.ShapeDtypeStruct((ER, 128), jnp.float32),
            jax.ShapeDtypeStruct((NC * C_PAD,), jnp.float32),
        ],
        mesh=plsc.VectorSubcoreMesh(core_axis_name="c", subcore_axis_name="s"),
        compiler_params=pltpu.CompilerParams(needs_layout_passes=False),
        scratch_types=[
            pltpu.VMEM((N,), jnp.float32),
            pltpu.VMEM((N,), jnp.float32),
            pltpu.VMEM((N,), jnp.float32),
            pltpu.VMEM((N,), jnp.float32),
            pltpu.VMEM((N,), jnp.float32),
            pltpu.VMEM((R,), jnp.float32),
            pltpu.VMEM((L,), jnp.float32),
            pltpu.VMEM((N_PAD,), jnp.float32),
            pltpu.VMEM((N_PAD,), jnp.float32),
            pltpu.VMEM((CHR, 128), jnp.int32),
            pltpu.VMEM((CHR, 128), jnp.int32),
            pltpu.VMEM((CHR, 128), jnp.int32),
            pltpu.VMEM((CHR, 128), jnp.int32),
            pltpu.VMEM((CHR, 128), jnp.int32),
            pltpu.VMEM((CHR, 128), jnp.float32),
            pltpu.VMEM((CHR, 128), jnp.float32),
            pltpu.VMEM((CHR, 128), jnp.float32),
            pltpu.VMEM((NSEG,), jnp.float32),
            pltpu.VMEM_SHARED((C_PAD,), jnp.float32),
        ],
    )
    return f(eh2, et2, rel2, tn2, ci2, u1_2, ntt, rtab, svec, sumA_p)


def _edge3_body(et_hbm, ci_hbm, v_hbm, xrt_hbm, sumB_hbm,
                acc_hbm,
                sb0_v, sb1_v, et_v, ci_v, v_v, w_v, rows_v, rows1_v, rows2_v,
                rows3_v, zz_v, acc_sp, sem, ssem):
    cid = lax.axis_index("c")
    sid = lax.axis_index("s")
    wid = sid * NC + cid
    pltpu.sync_copy(sumB_hbm.at[pl.ds(0, C_PAD)], sb0_v)
    pltpu.sync_copy(sumB_hbm.at[pl.ds(C_PAD, C_PAD)], sb1_v)

    def _zr(r, _):
        for l in range(8):
            zz_v[r, pl.ds(l * L, L)] = jnp.zeros((L,), jnp.float32)
        return 0
    lax.fori_loop(0, 16, _zr, 0)

    def _zc(i, _):
        pltpu.sync_copy(zz_v, acc_sp.at[pl.ds(sid * CSEG + i * 16, 16)])
        return 0
    lax.fori_loop(0, CSEG // 16, _zc, 0)
    plsc.subcore_barrier()

    nch = jnp.where(cid == FAST_CID, FCH, SCH)
    base_row = jnp.where(cid == FAST_CID, sid * FCH * CHR,
                         NS * FCH * CHR + sid * SCH * CHR)

    def _chunk(ch, _):
        r0 = base_row + ch * CHR
        pltpu.sync_copy(et_hbm.at[pl.ds(r0, CHR)], et_v)
        pltpu.sync_copy(ci_hbm.at[pl.ds(r0, CHR)], ci_v)
        pltpu.sync_copy(v_hbm.at[pl.ds(r0, CHR)], v_v)

        def _wrow(r, _):
            for l in range(8):
                cix = ci_v[r, pl.ds(l * L, L)]
                sb = (plsc.load_gather(sb0_v, [cix])
                      + plsc.load_gather(sb1_v, [cix]))
                w_v[r, pl.ds(l * L, L)] = v_v[r, pl.ds(l * L, L)] / (sb + 1e-38)
            return 0
        lax.fori_loop(0, CHR, _wrow, 0)

        def _scale_row(rows_b, r):
            def _sg(g, _):
                for i in range(L):
                    e = g * L + i
                    wb = plsc.load_gather(
                        w_v, [jnp.full((L,), r, jnp.int32),
                              jnp.full((L,), e, jnp.int32)])
                    for l in range(8):
                        rows_b[e, pl.ds(l * L, L)] = (
                            rows_b[e, pl.ds(l * L, L)] * wb)
                return 0
            lax.fori_loop(0, 128 // L, _sg, 0)

        bufs = (rows_v, rows1_v, rows2_v, rows3_v)

        def _drain(sm):
            pltpu.make_async_copy(xrt_hbm.at[pl.ds(0, 128)], rows_v, sm).wait()

        pltpu.async_copy(xrt_hbm.at[et_v.at[0]], bufs[0], sem)
        pltpu.async_copy(xrt_hbm.at[et_v.at[1]], bufs[1], sem)

        def _quad(q, _):
            for i in range(4):
                r = 4 * q + i

                @pl.when(r >= 2)
                def _drain_scat():
                    _drain(ssem)

                @pl.when(r + 2 < CHR)
                def _fire_gather():
                    pltpu.async_copy(xrt_hbm.at[et_v.at[r + 2]],
                                     bufs[(i + 2) % 4], sem)
                _drain(sem)
                _scale_row(bufs[i], r)
                pltpu.async_copy(bufs[i], acc_sp.at[ci_v.at[r]], ssem,
                                 add=True)
            return 0
        lax.fori_loop(0, CHR // 4, _quad, 0)
        _drain(ssem)
        _drain(ssem)
        return 0
    lax.fori_loop(0, nch, _chunk, 0)
    plsc.subcore_barrier()
    pltpu.sync_copy(acc_sp.at[pl.ds(sid * CSEG, CSEG)],
                    acc_hbm.at[pl.ds(cid * C_PAD + sid * CSEG, CSEG)])


def _edge3_call(et2, ci2, v2, x_r_t, sumB_p):
    f = pl.kernel(
        _edge3_body,
        out_type=[
            jax.ShapeDtypeStruct((NC * C_PAD, H), jnp.float32),
        ],
        mesh=plsc.VectorSubcoreMesh(core_axis_name="c", subcore_axis_name="s"),
        compiler_params=pltpu.CompilerParams(needs_layout_passes=False),
        scratch_types=[
            pltpu.VMEM((C_PAD,), jnp.float32),
            pltpu.VMEM((C_PAD,), jnp.float32),
            pltpu.VMEM((CHR, 128), jnp.int32),
            pltpu.VMEM((CHR, 128), jnp.int32),
            pltpu.VMEM((CHR, 128), jnp.float32),
            pltpu.VMEM((CHR, 128), jnp.float32),
            pltpu.VMEM((128, H), jnp.float32),
            pltpu.VMEM((128, H), jnp.float32),
            pltpu.VMEM((128, H), jnp.float32),
            pltpu.VMEM((128, H), jnp.float32),
            pltpu.VMEM((16, H), jnp.float32),
            pltpu.VMEM_SHARED((C_PAD, H), jnp.float32),
            pltpu.SemaphoreType.DMA,
            pltpu.SemaphoreType.DMA,
        ],
    )
    return f(et2, ci2, v2, x_r_t, sumB_p)[0]


CW = C_PAD // NW
CWG = CW // L


def _cls1_body(acc_hbm, hc_hbm, ntt_hbm, ac_hbm, svec_hbm,
               xcls_hbm, u3_hbm, sumC_hbm,
               q4_v, ac_v, sv_v, hc_v, hc1_v, u3_v, u3f_v, ec_v, xc_v, xc1_v,
               zz_v, sumC_sp):
    cid = lax.axis_index("c")
    sid = lax.axis_index("s")
    wid = sid * NC + cid
    pltpu.sync_copy(ntt_hbm.at[pl.ds(3 * N, N)], q4_v)
    pltpu.sync_copy(ac_hbm, ac_v)
    pltpu.sync_copy(svec_hbm.at[pl.ds(2 * L, L)], sv_v)
    pltpu.sync_copy(hc_hbm.at[pl.ds(wid * 256, 256)], hc1_v)

    def _hcb(g, _):
        hc_v[g] = hc1_v[pl.ds(g * L, L)]
        return 0
    lax.fori_loop(0, CWG, _hcb, 0)

    def _zf(i, _):
        zz_v[pl.ds(i * L, L)] = jnp.zeros((L,), jnp.float32)
        return 0
    lax.fori_loop(0, NSEG // L, _zf, 0)
    pltpu.sync_copy(zz_v, sumC_sp.at[pl.ds(sid * NSEG, NSEG)])
    plsc.subcore_barrier()

    pltpu.sync_copy(acc_hbm.at[pl.ds(wid * CW, CW)], xc_v)
    pltpu.sync_copy(acc_hbm.at[pl.ds(C_PAD + wid * CW, CW)], xc1_v)

    def _addr(r, _):
        for l in range(8):
            xc_v[r, pl.ds(l * L, L)] = (xc_v[r, pl.ds(l * L, L)]
                                        + xc1_v[r, pl.ds(l * L, L)])
        return 0
    lax.fori_loop(0, CW, _addr, 0)
    pltpu.sync_copy(xc_v, xcls_hbm.at[pl.ds(wid * CW, CW)])

    def _dotg(g, _):
        rows16 = g * L + lax.broadcasted_iota(jnp.int32, (L,), 0)

        def _j(j, acc):
            colv = plsc.load_gather(xc_v, [rows16, jnp.full((L,), j, jnp.int32)])
            acb = plsc.load_gather(ac_v, [jnp.full((L,), j, jnp.int32)])
            return acc + colv * acb
        ec_v[g] = lax.fori_loop(0, H, _j, jnp.zeros((L,), jnp.float32))
        return 0
    lax.fori_loop(0, CWG, _dotg, 0)

    S3 = sv_v[...]

    def _grp(g, _):
        hcx = hc_v[g]
        q4g = plsc.load_gather(q4_v, [jnp.minimum(hcx, N - 1)])
        e_c = ec_v[g] + q4g
        z = jnp.where(e_c >= 0, e_c, 0.01 * e_c)
        u3g = jnp.where(hcx < N, jnp.exp(z - S3), 0.0)
        u3_v[g] = u3g
        u3f_v[pl.ds(g * L, L)] = u3g
        pltpu.sync_copy(u3_v.at[g], sumC_sp.at[hc_v.at[g]], add=True)
        return 0
    lax.fori_loop(0, CWG, _grp, 0)
    pltpu.sync_copy(u3f_v, u3_hbm.at[pl.ds(wid * 256, 256)])
    plsc.subcore_barrier()
    pltpu.sync_copy(sumC_sp.at[pl.ds(sid * NSEG, NSEG)],
                    sumC_hbm.at[pl.ds(cid * N_PAD + sid * NSEG, NSEG)])


def _cls1_call(acc_p, hc2, ntt, a_c, svec):
    f = pl.kernel(
        _cls1_body,
        out_type=[
            jax.ShapeDtypeStruct((C_PAD, H), jnp.float32),
            jax.ShapeDtypeStruct((NW * 256,), jnp.float32),
            jax.ShapeDtypeStruct((NC * N_PAD,), jnp.float32),
        ],
        mesh=plsc.VectorSubcoreMesh(core_axis_name="c", subcore_axis_name="s"),
        compiler_params=pltpu.CompilerParams(needs_layout_passes=False),
        scratch_types=[
            pltpu.VMEM((N,), jnp.float32),
            pltpu.VMEM((H,), jnp.float32),
            pltpu.VMEM((L,), jnp.float32),
            pltpu.VMEM((L, L), jnp.int32),
            pltpu.VMEM((256,), jnp.int32),
            pltpu.VMEM((L, L), jnp.float32),
            pltpu.VMEM((256,), jnp.float32),
            pltpu.VMEM((L, L), jnp.float32),
            pltpu.VMEM((CW, H), jnp.float32),
            pltpu.VMEM((CW, H), jnp.float32),
            pltpu.VMEM((NSEG,), jnp.float32),
            pltpu.VMEM_SHARED((N_PAD,), jnp.float32),
        ],
    )
    return f(acc_p, hc2, ntt, a_c, svec)


def _cls2_body(xcls_hbm, u3_hbm, hc_hbm, sumC_hbm,
               xeh_hbm,
               sc0_v, sc1_v, hc_v, hc1_v, u3f_v, gm_v, xc_v, zz_v,
               xeh_sp):
    cid = lax.axis_index("c")
    sid = lax.axis_index("s")
    wid = sid * NC + cid
    pltpu.sync_copy(sumC_hbm.at[pl.ds(0, N_PAD)], sc0_v)
    pltpu.sync_copy(sumC_hbm.at[pl.ds(N_PAD, N_PAD)], sc1_v)
    pltpu.sync_copy(hc_hbm.at[pl.ds(wid * 256, 256)], hc1_v)
    pltpu.sync_copy(u3_hbm.at[pl.ds(wid * 256, 256)], u3f_v)
    pltpu.sync_copy(xcls_hbm.at[pl.ds(wid * CW, CW)], xc_v)

    def _hcb(g, _):
        hc_v[g] = hc1_v[pl.ds(g * L, L)]
        return 0
    lax.fori_loop(0, CWG, _hcb, 0)

    def _zr(r, _):
        for l in range(8):
            zz_v[r, pl.ds(l * L, L)] = jnp.zeros((L,), jnp.float32)
        return 0
    lax.fori_loop(0, 16, _zr, 0)

    def _zc(i, _):
        pltpu.sync_copy(zz_v, xeh_sp.at[pl.ds(sid * NSEG + i * 16, 16)])
        return 0
    lax.fori_loop(0, NSEG // 16, _zc, 0)
    plsc.subcore_barrier()

    def _grp(g, _):
        hcx = hc_v[g]
        sc = plsc.load_gather(sc0_v, [hcx]) + plsc.load_gather(sc1_v, [hcx])
        gm_v[g] = u3f_v[pl.ds(g * L, L)] / (sc + 1e-38)
        return 0
    lax.fori_loop(0, CWG, _grp, 0)

    def _scale(r, _):
        wb = plsc.load_gather(
            gm_v, [jnp.full((L,), r // L, jnp.int32),
                   jnp.full((L,), r % L, jnp.int32)])
        for l in range(8):
            xc_v[r, pl.ds(l * L, L)] = xc_v[r, pl.ds(l * L, L)] * wb
        return 0
    lax.fori_loop(0, CW, _scale, 0)

    def _scat(g, _):
        pltpu.sync_copy(xc_v.at[pl.ds(g * L, L)], xeh_sp.at[hc_v.at[g]],
                        add=True)
        return 0
    lax.fori_loop(0, CWG, _scat, 0)
    plsc.subcore_barrier()
    pltpu.sync_copy(xeh_sp.at[pl.ds(sid * NSEG, NSEG)],
                    xeh_hbm.at[pl.ds(cid * N_PAD + sid * NSEG, NSEG)])


def _cls2_call(xcls, u3, hc2, sumC_p):
    f = pl.kernel(
        _cls2_body,
        out_type=[
            jax.ShapeDtypeStruct((NC * N_PAD, H), jnp.float32),
        ],
        mesh=plsc.VectorSubcoreMesh(core_axis_name="c", subcore_axis_name="s"),
        compiler_params=pltpu.CompilerParams(needs_layout_passes=False),
        scratch_types=[
            pltpu.VMEM((N_PAD,), jnp.float32),
            pltpu.VMEM((N_PAD,), jnp.float32),
            pltpu.VMEM((L, L), jnp.int32),
            pltpu.VMEM((256,), jnp.int32),
            pltpu.VMEM((256,), jnp.float32),
            pltpu.VMEM((L, L), jnp.float32),
            pltpu.VMEM((CW, H), jnp.float32),
            pltpu.VMEM((16, H), jnp.float32),
            pltpu.VMEM_SHARED((N_PAD, H), jnp.float32),
        ],
    )
    return f(xcls, u3, hc2, sumC_p)[0]


def _gate_body(h_ref, e0_ref, e1_ref, w_ref, b_ref, o_ref):
    h = h_ref[...]
    g = jax.nn.sigmoid(jax.lax.dot(h, w_ref[...], preferred_element_type=jnp.float32)
                       + b_ref[...][None, :])
    o_ref[...] = g * (e0_ref[...] + e1_ref[...]) + (1.0 - g) * h


def _gate_call(x_r_h, xeh0, xeh1, hw_W, hw_b):
    return pl.pallas_call(
        _gate_body,
        grid=(N // NB,),
        in_specs=[
            pl.BlockSpec((NB, H), lambda i: (i, 0)),
            pl.BlockSpec((NB, H), lambda i: (i, 0)),
            pl.BlockSpec((NB, H), lambda i: (i, 0)),
            pl.BlockSpec((H, H), lambda i: (0, 0)),
            pl.BlockSpec((H,), lambda i: (0,)),
        ],
        out_specs=pl.BlockSpec((NB, H), lambda i: (i, 0)),
        out_shape=jax.ShapeDtypeStruct((N, H), jnp.float32),
    )(x_r_h, xeh0, xeh1, hw_W, hw_b)


def kernel(x_e, edge_index, rel, triple_num, r_emb, class_index, head_class,
           a_h1, a_h2, a_h3, a_h4, a_t1, a_t2, a_t3, a_r1, a_r2, a_c,
           W_h, W_t, hw_W, hw_b):
    Ah = jnp.stack([a_h1, a_h2, a_h3, a_h4], axis=1)
    At = jnp.stack([a_t1, a_t2, a_t3], axis=1)
    Ar = jnp.stack([a_r1, a_r2], axis=1)
    x_r_h, x_r_t, ntab = _nodes_call(x_e, W_h, W_t, Ah, At)
    rtab, svec, ntt = _rel_call(ntab, r_emb, Ar, a_t2, a_t3, a_c)
    S1, S2, S3 = svec[0, 0], svec[1, 0], svec[2, 0]
    q1, q2, q3, q4 = ntt[0], ntt[1], ntt[2], ntt[3]
    t1, t2, t3, nrm = ntt[4], ntt[5], ntt[6], ntt[7]
    r1, r2 = rtab[0], rtab[1]
    eh, et = edge_index[0], edge_index[1]
    tn, ci, hc = triple_num, class_index, head_class
    EPS = 1e-38

    pad_i = jnp.zeros((E_PAD - E,), jnp.int32)
    eh2 = jnp.concatenate([eh, pad_i]).reshape(ER, 128)
    et2 = jnp.concatenate([et, pad_i]).reshape(ER, 128)
    rel2 = jnp.concatenate([rel, pad_i]).reshape(ER, 128)
    tn2 = jnp.concatenate([tn, jnp.full((E_PAD - E,), N_PAD - 1, jnp.int32)]
                          ).reshape(ER, 128)
    ci2 = jnp.concatenate([ci, jnp.full((E_PAD - E,), C_PAD - 1, jnp.int32)]
                          ).reshape(ER, 128)
    hcp = jnp.concatenate([hc, jnp.full((C_PAD - C,), N_PAD - 1, jnp.int32)])
    hc3 = jnp.pad(hcp.reshape(NW, CW), ((0, 0), (0, 256 - CW)),
                  constant_values=N_PAD - 1).reshape(NW * 256)
    ntt_f = ntt.reshape(8 * N)
    rtab_f = rtab.reshape(2 * R)
    svec_f = svec.reshape(8 * L)

    u1_2, sumA_p = _edge1_call(eh2, et2, rel2, tn2, ntt_f, rtab_f, svec_f)
    v2, sumB_p = _edge2_call(eh2, et2, rel2, tn2, ci2, u1_2, ntt_f, rtab_f,
                             svec_f, sumA_p)
    acc_p = _edge3_call(et2, ci2, v2, x_r_t, sumB_p)
    xcls, u3, sumC_p = _cls1_call(acc_p, hc3, ntt_f, a_c, svec_f)
    xeh_p = _cls2_call(xcls, u3, hc3, sumC_p)

    return _gate_call(x_r_h, xeh_p[:N], xeh_p[N_PAD:N_PAD + N], hw_W, hw_b)
